# Initial kernel scaffold; baseline (speedup 1.0000x reference)
import jax
import jax.numpy as jnp
from jax.experimental import pallas as pl


def kernel(x_nchw, p000, p001, p002, p003, p004, p005, p006, p007, p008, p009, p010, p011, p012, p013, p014, p015, p016, p017, p018, p019, p020, p021, p022, p023, p024, p025, p026, p027, p028, p029, p030, p031, p032, p033, p034, p035, p036, p037, p038, p039, p040, p041, p042, p043, p044, p045, p046, p047, p048, p049, p050, p051, p052, p053, p054, p055, p056, p057, p058, p059, p060, p061, p062, p063, p064, p065, p066, p067, p068, p069, p070, p071, p072, p073, p074, p075, p076, p077, p078, p079, p080, p081, p082, p083, p084, p085, p086, p087, p088, p089, p090, p091, p092, p093, p094, p095, p096, p097, p098, p099, p100, p101, p102, p103, p104, p105, p106, p107, p108, p109, p110, p111, p112, p113, p114, p115, p116, p117, p118, p119, p120, p121, p122, p123, p124, p125, p126, p127, p128, p129, p130, p131, p132, p133, p134, p135, p136, p137, p138, p139, p140, p141, p142, p143, p144, p145, p146, p147, p148, p149, p150, p151, p152, p153, p154, p155, p156, p157, p158, p159, p160, p161, p162, p163, p164, p165):
    raise NotImplementedError("write your pallas kernel here")


# speedup vs baseline: 1.0473x; 1.0473x over previous
import functools
import numpy as np
import jax
import jax.numpy as jnp
from jax import lax
from jax.experimental import pallas as pl
from jax.experimental.pallas import tpu as pltpu

_VMEM_LIMIT = 44 * 2**20


def _pick_tile(dim, cap, align):
    if dim <= cap:
        return dim
    t = (cap // align) * align
    while t >= align:
        if dim % t == 0:
            return t
        t -= align
    return dim


def _mm_kernel(a_ref, b_ref, bias_ref, o_ref, sum_ref, sq_ref):
    y = jnp.dot(a_ref[...], b_ref[...], preferred_element_type=jnp.float32)
    y = y + bias_ref[...]
    o_ref[...] = y.astype(o_ref.dtype)
    sum_ref[...] = jnp.sum(y, axis=0, keepdims=True)[None, :, :]
    sq_ref[...] = jnp.sum(y * y, axis=0, keepdims=True)[None, :, :]


def _mm_kernel_kt(a_ref, b_ref, bias_ref, o_ref, sum_ref, sq_ref, acc_ref):
    k = pl.program_id(2)

    @pl.when(k == 0)
    def _init():
        acc_ref[...] = jnp.zeros_like(acc_ref)

    acc_ref[...] += jnp.dot(a_ref[...], b_ref[...],
                            preferred_element_type=jnp.float32)

    @pl.when(k == pl.num_programs(2) - 1)
    def _store():
        y = acc_ref[...] + bias_ref[...]
        o_ref[...] = y.astype(o_ref.dtype)
        sum_ref[...] = jnp.sum(y, axis=0, keepdims=True)[None, :, :]
        sq_ref[...] = jnp.sum(y * y, axis=0, keepdims=True)[None, :, :]


def matmul_bias_stats(a, b, bias, out_dtype=jnp.bfloat16):
    M, K = a.shape
    Kb, N = b.shape
    assert K == Kb
    a = a.astype(jnp.bfloat16)
    b = b.astype(jnp.bfloat16)
    bias = bias.astype(jnp.float32)

    if K % 8 != 0:
        Kp = ((K + 127) // 128) * 128
        a = jnp.pad(a, ((0, 0), (0, Kp - K)))
        b = jnp.pad(b, ((0, Kp - K), (0, 0)))
        K = Kp
    n_orig = N
    if N % 8 != 0:
        Np = ((N + 127) // 128) * 128
        b = jnp.pad(b, ((0, 0), (0, Np - N)))
        bias = jnp.pad(bias, ((0, Np - N),))
        N = Np

    tm = _pick_tile(M, 2048, 16 if M % 16 == 0 else 8)
    tn = N if N <= 512 else _pick_tile(N, 512, 128)

    panel_bytes = 2 * (tm + tn) * K
    if panel_bytes <= 24 * 2**20:
        nmt, nnt = M // tm, N // tn
        out, psum, psq = pl.pallas_call(
            _mm_kernel,
            out_shape=(jax.ShapeDtypeStruct((M, N), out_dtype),
                       jax.ShapeDtypeStruct((nmt, 1, N), jnp.float32),
                       jax.ShapeDtypeStruct((nmt, 1, N), jnp.float32)),
            grid=(nmt, nnt),
            in_specs=[pl.BlockSpec((tm, K), lambda i, j: (i, 0)),
                      pl.BlockSpec((K, tn), lambda i, j: (0, j)),
                      pl.BlockSpec((1, tn), lambda i, j: (0, j))],
            out_specs=(pl.BlockSpec((tm, tn), lambda i, j: (i, j)),
                       pl.BlockSpec((1, 1, tn), lambda i, j: (i, 0, j)),
                       pl.BlockSpec((1, 1, tn), lambda i, j: (i, 0, j))),
            compiler_params=pltpu.CompilerParams(
                dimension_semantics=("parallel", "parallel"),
                vmem_limit_bytes=_VMEM_LIMIT),
        )(a, b, bias.reshape(1, N))
    else:
        tk = _pick_tile(K, 2048, 128)
        nmt, nnt, nkt = M // tm, N // tn, K // tk
        out, psum, psq = pl.pallas_call(
            _mm_kernel_kt,
            out_shape=(jax.ShapeDtypeStruct((M, N), out_dtype),
                       jax.ShapeDtypeStruct((nmt, 1, N), jnp.float32),
                       jax.ShapeDtypeStruct((nmt, 1, N), jnp.float32)),
            grid=(nmt, nnt, nkt),
            in_specs=[pl.BlockSpec((tm, tk), lambda i, j, k: (i, k)),
                      pl.BlockSpec((tk, tn), lambda i, j, k: (k, j)),
                      pl.BlockSpec((1, tn), lambda i, j, k: (0, j))],
            out_specs=(pl.BlockSpec((tm, tn), lambda i, j, k: (i, j)),
                       pl.BlockSpec((1, 1, tn), lambda i, j, k: (i, 0, j)),
                       pl.BlockSpec((1, 1, tn), lambda i, j, k: (i, 0, j))),
            scratch_shapes=[pltpu.VMEM((tm, tn), jnp.float32)],
            compiler_params=pltpu.CompilerParams(
                dimension_semantics=("parallel", "parallel", "arbitrary"),
                vmem_limit_bytes=_VMEM_LIMIT),
        )(a, b, bias.reshape(1, N))

    col_sum = jnp.sum(psum[:, 0, :], axis=0)
    col_sq = jnp.sum(psq[:, 0, :], axis=0)
    if n_orig != N:
        out = out[:, :n_orig]
        col_sum = col_sum[:n_orig]
        col_sq = col_sq[:n_orig]
    return out, col_sum, col_sq


def _bn_kernel(x_ref, scale_ref, shift_ref, o_ref, *, relu):
    y = x_ref[...].astype(jnp.float32) * scale_ref[...] + shift_ref[...]
    if relu:
        y = jnp.maximum(y, 0.0)
    o_ref[...] = y.astype(o_ref.dtype)


def _bn_res_kernel(x_ref, scale_ref, shift_ref, r_ref, o_ref, *, relu):
    y = (x_ref[...].astype(jnp.float32) * scale_ref[...] + shift_ref[...]
         + r_ref[...].astype(jnp.float32))
    if relu:
        y = jnp.maximum(y, 0.0)
    o_ref[...] = y.astype(o_ref.dtype)


def _bn_post_kernel(x_ref, scale_ref, shift_ref, r_ref, o_ref, *, relu):
    y = x_ref[...].astype(jnp.float32) * scale_ref[...] + shift_ref[...]
    if relu:
        y = jnp.maximum(y, 0.0)
    o_ref[...] = (y + r_ref[...].astype(jnp.float32)).astype(o_ref.dtype)


def _bn_scale_shift(p, stats, C, eps=1e-5):
    s, ss, cnt = stats
    mean = (s / cnt).reshape(1, C)
    var = jnp.maximum(ss / cnt - mean * mean, 0.0)
    scale = p['gamma'].reshape(1, C) * lax.rsqrt(var + eps)
    shift = p['beta'].reshape(1, C) - mean * scale
    return scale, shift


def batchnorm2d(x, p, stats, relu=False, residual=None, post_add=None,
                eps=1e-5, out_dtype=jnp.bfloat16):
    N, H, W, C = x.shape
    M = N * H * W
    scale, shift = _bn_scale_shift(p, stats, C, eps)

    extra = residual if residual is not None else post_add

    f = 1
    if C < 128 and 128 % C == 0 and M % (128 // C) == 0:
        f = 128 // C
    Mp, Cp = M // f, C * f
    x2 = x.reshape(Mp, Cp)
    scale_p = jnp.tile(scale, (1, f))
    shift_p = jnp.tile(shift, (1, f))

    tm = _pick_tile(Mp, 4096, 8)

    in_specs = [pl.BlockSpec((tm, Cp), lambda i: (i, 0)),
                pl.BlockSpec((1, Cp), lambda i: (0, 0)),
                pl.BlockSpec((1, Cp), lambda i: (0, 0))]
    args = [x2, scale_p, shift_p]
    if extra is not None:
        in_specs.append(pl.BlockSpec((tm, Cp), lambda i: (i, 0)))
        args.append(extra.reshape(Mp, Cp))
        kern = functools.partial(
            _bn_res_kernel if residual is not None else _bn_post_kernel, relu=relu)
    else:
        kern = functools.partial(_bn_kernel, relu=relu)

    out = pl.pallas_call(
        kern,
        out_shape=jax.ShapeDtypeStruct((Mp, Cp), out_dtype),
        grid=(Mp // tm,),
        in_specs=in_specs,
        out_specs=pl.BlockSpec((tm, Cp), lambda i: (i, 0)),
        compiler_params=pltpu.CompilerParams(
            dimension_semantics=("parallel",),
            vmem_limit_bytes=_VMEM_LIMIT),
    )(*args)
    return out.reshape(N, H, W, C)


def _im2col(x, kh, kw, stride, pt, pb, pl_, pr):
    x = jnp.pad(x, ((0, 0), (pt, pb), (pl_, pr), (0, 0)))
    N, H, W, C = x.shape
    Ho = (H - kh) // stride + 1
    Wo = (W - kw) // stride + 1
    cols = []
    for i in range(kh):
        for j in range(kw):
            cols.append(x[:, i:i + (Ho - 1) * stride + 1:stride,
                            j:j + (Wo - 1) * stride + 1:stride, :])
    patches = jnp.stack(cols, axis=-2)
    return patches.reshape(N * Ho * Wo, kh * kw * C), (N, Ho, Wo)


def conv2d(x, p, stride=1, padding=(0, 0), out_dtype=jnp.bfloat16):
    w, b = p['w'], p['b']
    cout, cin, kh, kw = w.shape
    N, H, W, _ = x.shape
    if kh == 1 and kw == 1 and stride == 1:
        mat = x.reshape(N * H * W, cin)
        wmat = jnp.transpose(w, (2, 3, 1, 0)).reshape(cin, cout)
        out, s, ss = matmul_bias_stats(mat, wmat, b, out_dtype=out_dtype)
        return out.reshape(N, H, W, cout), (s, ss, N * H * W)
    patches, (N, Ho, Wo) = _im2col(x.astype(jnp.bfloat16), kh, kw, stride,
                                   padding[0], padding[0], padding[1], padding[1])
    wmat = jnp.transpose(w, (2, 3, 1, 0)).reshape(kh * kw * cin, cout)
    out, s, ss = matmul_bias_stats(patches, wmat, b, out_dtype=out_dtype)
    return out.reshape(N, Ho, Wo, cout), (s, ss, N * Ho * Wo)


def _s2_taps(k, pad):
    taps = {0: [], 1: []}
    for d in (0, 1):
        for i in range(k):
            if (d + pad - i) % 2 == 0:
                taps[d].append(((d + pad - i) // 2, i))
    ts = sorted({t for d in (0, 1) for (t, _) in taps[d]})
    return taps, ts


def conv_transpose2d(x, p, padding=(0, 0), output_padding=(0, 0),
                     out_dtype=jnp.bfloat16):
    w, b = p['w'], p['b']
    cin, cout, kh, kw = w.shape
    N, H, W, _ = x.shape
    Ho = 2 * H
    Wo = 2 * W

    taps_h, ts_h = _s2_taps(kh, padding[0])
    taps_w, ts_w = _s2_taps(kw, padding[1])
    ph_lo, ph_hi = max(0, -ts_h[0]), max(0, ts_h[-1])
    pw_lo, pw_hi = max(0, -ts_w[0]), max(0, ts_w[-1])

    if len(ts_h) == 1 and len(ts_w) == 1 and ts_h[0] == 0 and ts_w[0] == 0:
        patches = x.astype(jnp.bfloat16).reshape(N * H * W, cin)
    else:
        xp = jnp.pad(x.astype(jnp.bfloat16),
                     ((0, 0), (ph_lo, ph_hi), (pw_lo, pw_hi), (0, 0)))
        cols = []
        for th in ts_h:
            for tw in ts_w:
                cols.append(xp[:, th + ph_lo: th + ph_lo + H,
                                 tw + pw_lo: tw + pw_lo + W, :])
        patches = jnp.concatenate(cols, axis=-1).reshape(
            N * H * W, len(ts_h) * len(ts_w) * cin)

    ih_tbl = np.full((len(ts_h), 2), -1, np.int64)
    for d in (0, 1):
        for (t, i) in taps_h[d]:
            ih_tbl[ts_h.index(t), d] = i
    iw_tbl = np.full((len(ts_w), 2), -1, np.int64)
    for d in (0, 1):
        for (t, i) in taps_w[d]:
            iw_tbl[ts_w.index(t), d] = i

    zeros = jnp.zeros((cin, cout), w.dtype)
    rows = []
    for a_ in range(len(ts_h)):
        for b_ in range(len(ts_w)):
            phase_cols = []
            for dh in (0, 1):
                for dw in (0, 1):
                    ih, iw = int(ih_tbl[a_, dh]), int(iw_tbl[b_, dw])
                    phase_cols.append(zeros if (ih < 0 or iw < 0) else w[:, :, ih, iw])
            rows.append(jnp.concatenate(phase_cols, axis=1))
    wmat = jnp.concatenate(rows, axis=0)

    out, s, ss = matmul_bias_stats(patches, wmat, jnp.tile(b, 4),
                                   out_dtype=out_dtype)
    out = out.reshape(N, H, W, 2, 2, cout)
    out = out.transpose(0, 1, 3, 2, 4, 5).reshape(N, 2 * H, 2 * W, cout)
    stats = (s.reshape(4, cout).sum(0), ss.reshape(4, cout).sum(0), N * Ho * Wo)
    return out, stats


def maxpool_3x3_s2_p1(x):
    xp = jnp.pad(x, ((0, 0), (1, 1), (1, 1), (0, 0)), constant_values=-jnp.inf)
    return lax.reduce_window(xp, -jnp.inf, lax.max, (1, 3, 3, 1), (1, 2, 2, 1),
                             'VALID')


def upsample_nearest(x, scale):
    if scale == 1:
        return x
    return jnp.repeat(jnp.repeat(x, scale, axis=1), scale, axis=2)


def conv_bn(x, cp, bnp, stride=1, padding=(0, 0), relu=True,
            residual=None, post_add=None):
    y, stats = conv2d(x, cp, stride=stride, padding=padding)
    return batchnorm2d(y, bnp, stats, relu=relu, residual=residual,
                       post_add=post_add)


def convT_bn(x, cp, bnp, padding=(0, 0), output_padding=(0, 0)):
    y, stats = conv_transpose2d(x, cp, padding=padding,
                                output_padding=output_padding)
    return batchnorm2d(y, bnp, stats, relu=True)


def basic_block(x, p, post_add=None):
    x = conv_bn(x, p['conv1'], p['bn1'], padding=(1, 1), relu=True)
    x = conv_bn(x, p['conv2'], p['bn2'], padding=(1, 1), relu=True,
                post_add=post_add)
    return x


def resnet_block(x, p, stride=1):
    identity = x
    out = conv_bn(x, p['conv1'], p['bn1'], stride=stride, padding=(1, 1),
                  relu=True)
    y, stats = conv2d(out, p['conv2'], stride=1, padding=(1, 1))
    if 'down_conv' in p:
        identity = conv_bn(x, p['down_conv'], p['down_bn'],
                           stride=stride, padding=(0, 0), relu=False)
    return batchnorm2d(y, p['bn2'], stats, relu=True, residual=identity)


def encoder(x, p):
    x = conv_bn(x, p['conv1'], p['bn1'], stride=2, padding=(3, 3), relu=True)
    x = maxpool_3x3_s2_p1(x)
    x = resnet_block(x, p['layer1'][0])
    x = resnet_block(x, p['layer1'][1])
    x1 = x
    x = resnet_block(x, p['layer2'][0], stride=2)
    x = resnet_block(x, p['layer2'][1])
    x2 = x
    x = resnet_block(x, p['layer3'][0], stride=2)
    x = resnet_block(x, p['layer3'][1])
    x3 = x
    x = resnet_block(x, p['layer4'][0], stride=2)
    x = resnet_block(x, p['layer4'][1])
    x4 = x
    return x1, x2, x3, x4


def upsample_module(x, p, padding=(0, 0), output_padding=(0, 0), skip=None):
    x = convT_bn(x, p['convT'], p['bn'], padding, output_padding)
    x = basic_block(x, p['conv'], post_add=skip)
    return x


def fpn_module(x, p, scale):
    x = conv_bn(x, p['conv1'], p['bn'], padding=(1, 1), relu=True)
    return upsample_nearest(x, scale)


def row_detection(x, p):
    x1, x2, x3, x4 = encoder(x, p['down'])
    x = upsample_module(x4, p['up1'], padding=(1, 1), output_padding=(1, 1),
                        skip=x3)
    x3 = x
    x = upsample_module(x, p['up2'], padding=(1, 1), output_padding=(1, 1),
                        skip=x2)
    x2 = x
    x = upsample_module(x, p['up3'], padding=(1, 1), output_padding=(1, 1),
                        skip=x1)
    x1 = x
    x = upsample_module(x, p['up4'], padding=(1, 1), output_padding=(1, 1))
    x3 = fpn_module(x3, p['fpn1'], 8)
    x2 = fpn_module(x2, p['fpn2'], 4)
    x1 = fpn_module(x1, p['fpn3'], 2)
    x = fpn_module(x, p['fpn4'], 1)
    x = jnp.concatenate([x, x1, x2, x3], axis=-1)
    x = upsample_module(x, p['up5'])
    x = basic_block(x, p['conv1'])
    y, _ = conv2d(x, p['conv2'], padding=(0, 0), out_dtype=jnp.float32)
    return y


class _ParamGen:
    def __init__(self, seed=0):
        self.key = jax.random.PRNGKey(seed)

    def next(self):
        self.key, sub = jax.random.split(self.key)
        return sub


def _make_conv(pg, cin, cout, k, bias=True):
    kh, kw = (k, k) if isinstance(k, int) else k
    w = jax.random.normal(pg.next(), (cout, cin, kh, kw), jnp.float32) / np.sqrt(cin * kh * kw)
    b = (jax.random.normal(pg.next(), (cout,), jnp.float32) * 0.01 if bias
         else jnp.zeros((cout,), jnp.float32))
    return {'w': w, 'b': b}


def _make_convT(pg, cin, cout, k):
    kh, kw = k
    w = jax.random.normal(pg.next(), (cin, cout, kh, kw), jnp.float32) / np.sqrt(cin * kh * kw)
    b = jax.random.normal(pg.next(), (cout,), jnp.float32) * 0.01
    return {'w': w, 'b': b}


def _make_bn(c):
    return {'gamma': jnp.ones((c,), jnp.float32), 'beta': jnp.zeros((c,), jnp.float32)}


def _make_resnet_block(pg, cin, cout, stride=1):
    p = {'conv1': _make_conv(pg, cin, cout, 3, bias=False), 'bn1': _make_bn(cout),
         'conv2': _make_conv(pg, cout, cout, 3, bias=False), 'bn2': _make_bn(cout)}
    if stride != 1 or cin != cout:
        p['down_conv'] = _make_conv(pg, cin, cout, 1, bias=False)
        p['down_bn'] = _make_bn(cout)
    return p


def _make_basic_block(pg, cin, cout):
    return {'conv1': _make_conv(pg, cin, cout, 3), 'bn1': _make_bn(cout),
            'conv2': _make_conv(pg, cout, cout, 3), 'bn2': _make_bn(cout)}


def _make_upsample(pg, cin, cout, k=(3, 3)):
    return {'convT': _make_convT(pg, cin, cout, k), 'bn': _make_bn(cout),
            'conv': _make_basic_block(pg, cout, cout)}


def _make_fpn(pg, cin):
    return {'conv1': _make_conv(pg, cin, 64, 3), 'bn': _make_bn(64)}


def _make_params(out_channel=2):
    pg = _ParamGen(0)
    down = {'conv1': _make_conv(pg, 3, 64, 7, bias=False), 'bn1': _make_bn(64),
            'layer1': [_make_resnet_block(pg, 64, 64), _make_resnet_block(pg, 64, 64)],
            'layer2': [_make_resnet_block(pg, 64, 128, 2), _make_resnet_block(pg, 128, 128)],
            'layer3': [_make_resnet_block(pg, 128, 256, 2), _make_resnet_block(pg, 256, 256)],
            'layer4': [_make_resnet_block(pg, 256, 512, 2), _make_resnet_block(pg, 512, 512)]}
    return {'down': down,
            'up1': _make_upsample(pg, 512, 256), 'up2': _make_upsample(pg, 256, 128),
            'up3': _make_upsample(pg, 128, 64), 'up4': _make_upsample(pg, 64, 64),
            'fpn1': _make_fpn(pg, 256), 'fpn2': _make_fpn(pg, 128),
            'fpn3': _make_fpn(pg, 64), 'fpn4': _make_fpn(pg, 64),
            'up5': _make_upsample(pg, 256, 64, k=(2, 2)),
            'conv1': _make_basic_block(pg, 64, 32),
            'conv2': _make_conv(pg, 32, out_channel, 1)}


_TREEDEF = None


def _treedef():
    global _TREEDEF
    if _TREEDEF is None:
        _, _TREEDEF = jax.tree_util.tree_flatten(_make_params(2))
    return _TREEDEF


def kernel(x_nchw, *leaves):
    params = jax.tree_util.tree_unflatten(_treedef(), list(leaves))
    x = jnp.transpose(x_nchw, (0, 2, 3, 1))
    y = row_detection(x, params)
    return jnp.transpose(y, (0, 3, 1, 2))


# speedup vs baseline: 1.9875x; 1.8977x over previous
import functools
import numpy as np
import jax
import jax.numpy as jnp
from jax import lax
from jax.experimental import pallas as pl
from jax.experimental.pallas import tpu as pltpu

_VMEM_LIMIT = 44 * 2**20


def _pick_tile(dim, cap, align):
    if dim <= cap:
        return dim
    t = (cap // align) * align
    while t >= align:
        if dim % t == 0:
            return t
        t -= align
    return dim


def _mm_kernel(a_ref, b_ref, bias_ref, o_ref, sum_ref, sq_ref):
    y = jnp.dot(a_ref[...], b_ref[...], preferred_element_type=jnp.float32)
    y = y + bias_ref[...]
    o_ref[...] = y.astype(o_ref.dtype)
    sum_ref[...] = jnp.sum(y, axis=0, keepdims=True)[None, :, :]
    sq_ref[...] = jnp.sum(y * y, axis=0, keepdims=True)[None, :, :]


def _mm_kernel_kt(a_ref, b_ref, bias_ref, o_ref, sum_ref, sq_ref, acc_ref):
    k = pl.program_id(2)

    @pl.when(k == 0)
    def _init():
        acc_ref[...] = jnp.zeros_like(acc_ref)

    acc_ref[...] += jnp.dot(a_ref[...], b_ref[...],
                            preferred_element_type=jnp.float32)

    @pl.when(k == pl.num_programs(2) - 1)
    def _store():
        y = acc_ref[...] + bias_ref[...]
        o_ref[...] = y.astype(o_ref.dtype)
        sum_ref[...] = jnp.sum(y, axis=0, keepdims=True)[None, :, :]
        sq_ref[...] = jnp.sum(y * y, axis=0, keepdims=True)[None, :, :]


def matmul_bias_stats(a, b, bias, out_dtype=jnp.bfloat16):
    M, K = a.shape
    Kb, N = b.shape
    assert K == Kb
    a = a.astype(jnp.bfloat16)
    b = b.astype(jnp.bfloat16)
    bias = bias.astype(jnp.float32)

    if K % 8 != 0:
        Kp = ((K + 127) // 128) * 128
        a = jnp.pad(a, ((0, 0), (0, Kp - K)))
        b = jnp.pad(b, ((0, Kp - K), (0, 0)))
        K = Kp
    n_orig = N
    if N % 8 != 0:
        Np = ((N + 127) // 128) * 128
        b = jnp.pad(b, ((0, 0), (0, Np - N)))
        bias = jnp.pad(bias, ((0, Np - N),))
        N = Np

    tm = _pick_tile(M, 2048, 16 if M % 16 == 0 else 8)
    tn = N if N <= 512 else _pick_tile(N, 512, 128)

    panel_bytes = 2 * (tm + tn) * K
    if panel_bytes <= 24 * 2**20:
        nmt, nnt = M // tm, N // tn
        out, psum, psq = pl.pallas_call(
            _mm_kernel,
            out_shape=(jax.ShapeDtypeStruct((M, N), out_dtype),
                       jax.ShapeDtypeStruct((nmt, 1, N), jnp.float32),
                       jax.ShapeDtypeStruct((nmt, 1, N), jnp.float32)),
            grid=(nmt, nnt),
            in_specs=[pl.BlockSpec((tm, K), lambda i, j: (i, 0)),
                      pl.BlockSpec((K, tn), lambda i, j: (0, j)),
                      pl.BlockSpec((1, tn), lambda i, j: (0, j))],
            out_specs=(pl.BlockSpec((tm, tn), lambda i, j: (i, j)),
                       pl.BlockSpec((1, 1, tn), lambda i, j: (i, 0, j)),
                       pl.BlockSpec((1, 1, tn), lambda i, j: (i, 0, j))),
            compiler_params=pltpu.CompilerParams(
                dimension_semantics=("parallel", "parallel"),
                vmem_limit_bytes=_VMEM_LIMIT),
        )(a, b, bias.reshape(1, N))
    else:
        tk = _pick_tile(K, 2048, 128)
        nmt, nnt, nkt = M // tm, N // tn, K // tk
        out, psum, psq = pl.pallas_call(
            _mm_kernel_kt,
            out_shape=(jax.ShapeDtypeStruct((M, N), out_dtype),
                       jax.ShapeDtypeStruct((nmt, 1, N), jnp.float32),
                       jax.ShapeDtypeStruct((nmt, 1, N), jnp.float32)),
            grid=(nmt, nnt, nkt),
            in_specs=[pl.BlockSpec((tm, tk), lambda i, j, k: (i, k)),
                      pl.BlockSpec((tk, tn), lambda i, j, k: (k, j)),
                      pl.BlockSpec((1, tn), lambda i, j, k: (0, j))],
            out_specs=(pl.BlockSpec((tm, tn), lambda i, j, k: (i, j)),
                       pl.BlockSpec((1, 1, tn), lambda i, j, k: (i, 0, j)),
                       pl.BlockSpec((1, 1, tn), lambda i, j, k: (i, 0, j))),
            scratch_shapes=[pltpu.VMEM((tm, tn), jnp.float32)],
            compiler_params=pltpu.CompilerParams(
                dimension_semantics=("parallel", "parallel", "arbitrary"),
                vmem_limit_bytes=_VMEM_LIMIT),
        )(a, b, bias.reshape(1, N))

    col_sum = jnp.sum(psum[:, 0, :], axis=0)
    col_sq = jnp.sum(psq[:, 0, :], axis=0)
    if n_orig != N:
        out = out[:, :n_orig]
        col_sum = col_sum[:n_orig]
        col_sq = col_sq[:n_orig]
    return out, col_sum, col_sq


def _bn_kernel(x_ref, scale_ref, shift_ref, o_ref, *, relu):
    y = x_ref[...].astype(jnp.float32) * scale_ref[...] + shift_ref[...]
    if relu:
        y = jnp.maximum(y, 0.0)
    o_ref[...] = y.astype(o_ref.dtype)


def _bn_res_kernel(x_ref, scale_ref, shift_ref, r_ref, o_ref, *, relu):
    y = (x_ref[...].astype(jnp.float32) * scale_ref[...] + shift_ref[...]
         + r_ref[...].astype(jnp.float32))
    if relu:
        y = jnp.maximum(y, 0.0)
    o_ref[...] = y.astype(o_ref.dtype)


def _bn_post_kernel(x_ref, scale_ref, shift_ref, r_ref, o_ref, *, relu):
    y = x_ref[...].astype(jnp.float32) * scale_ref[...] + shift_ref[...]
    if relu:
        y = jnp.maximum(y, 0.0)
    o_ref[...] = (y + r_ref[...].astype(jnp.float32)).astype(o_ref.dtype)


def _bn_scale_shift(p, stats, C, eps=1e-5):
    s, ss, cnt = stats
    mean = (s / cnt).reshape(1, C)
    var = jnp.maximum(ss / cnt - mean * mean, 0.0)
    scale = p['gamma'].reshape(1, C) * lax.rsqrt(var + eps)
    shift = p['beta'].reshape(1, C) - mean * scale
    return scale, shift


def batchnorm2d(x, p, stats, relu=False, residual=None, post_add=None,
                eps=1e-5, out_dtype=jnp.bfloat16):
    N, H, W, C = x.shape
    M = N * H * W
    scale, shift = _bn_scale_shift(p, stats, C, eps)

    extra = residual if residual is not None else post_add

    f = 1
    if C < 128 and 128 % C == 0 and M % (128 // C) == 0:
        f = 128 // C
    Mp, Cp = M // f, C * f
    x2 = x.reshape(Mp, Cp)
    scale_p = jnp.tile(scale, (1, f))
    shift_p = jnp.tile(shift, (1, f))

    tm = _pick_tile(Mp, 4096, 8)

    in_specs = [pl.BlockSpec((tm, Cp), lambda i: (i, 0)),
                pl.BlockSpec((1, Cp), lambda i: (0, 0)),
                pl.BlockSpec((1, Cp), lambda i: (0, 0))]
    args = [x2, scale_p, shift_p]
    if extra is not None:
        in_specs.append(pl.BlockSpec((tm, Cp), lambda i: (i, 0)))
        args.append(extra.reshape(Mp, Cp))
        kern = functools.partial(
            _bn_res_kernel if residual is not None else _bn_post_kernel, relu=relu)
    else:
        kern = functools.partial(_bn_kernel, relu=relu)

    out = pl.pallas_call(
        kern,
        out_shape=jax.ShapeDtypeStruct((Mp, Cp), out_dtype),
        grid=(Mp // tm,),
        in_specs=in_specs,
        out_specs=pl.BlockSpec((tm, Cp), lambda i: (i, 0)),
        compiler_params=pltpu.CompilerParams(
            dimension_semantics=("parallel",),
            vmem_limit_bytes=_VMEM_LIMIT),
    )(*args)
    return out.reshape(N, H, W, C)


def _dconv3_kernel(a_ref, w_ref, b_ref, o_ref, sum_ref, sq_ref, acc_ref, *, th):
    Nb = a_ref.shape[0]
    W = a_ref.shape[3]
    C3 = a_ref.shape[4]
    Co = o_ref.shape[4]
    for i in range(3):
        l = a_ref[:, 0, i:i + th, :, :].reshape(Nb * th * W, C3)
        d = jnp.dot(l, w_ref[i], preferred_element_type=jnp.float32)
        if i == 0:
            acc_ref[...] = d
        else:
            acc_ref[...] += d
    y = acc_ref[...] + b_ref[...]
    o_ref[...] = y.astype(o_ref.dtype).reshape(Nb, 1, th, W, Co)
    sum_ref[...] = jnp.sum(y, axis=0, keepdims=True)[None, :, :]
    sq_ref[...] = jnp.sum(y * y, axis=0, keepdims=True)[None, :, :]


def conv3x3_direct(x, p, out_dtype=jnp.bfloat16):
    w, b = p['w'], p['b']
    cout, cin, _, _ = w.shape
    N, H, W, C = x.shape
    th = H if H <= 32 else 16
    HT = H // th

    xp = jnp.pad(x.astype(jnp.bfloat16), ((0, 0), (1, 1), (1, 1), (0, 0)))
    aw = jnp.concatenate([xp[:, :, 0:W, :], xp[:, :, 1:W + 1, :],
                          xp[:, :, 2:W + 2, :]], axis=-1)
    if HT == 1:
        a = aw[:, None]
    else:
        a = jnp.concatenate(
            [aw[:, h * th:h * th + th + 2][:, None] for h in range(HT)], axis=1)

    wm = jnp.transpose(w, (2, 3, 1, 0)).reshape(3, 3 * cin, cout).astype(jnp.bfloat16)
    bm = b.astype(jnp.float32).reshape(1, cout)
    M = N * th * W

    out, psum, psq = pl.pallas_call(
        functools.partial(_dconv3_kernel, th=th),
        out_shape=(jax.ShapeDtypeStruct((N, HT, th, W, cout), out_dtype),
                   jax.ShapeDtypeStruct((HT, 1, cout), jnp.float32),
                   jax.ShapeDtypeStruct((HT, 1, cout), jnp.float32)),
        grid=(HT,),
        in_specs=[pl.BlockSpec((N, 1, th + 2, W, 3 * C), lambda h: (0, h, 0, 0, 0)),
                  pl.BlockSpec((3, 3 * cin, cout), lambda h: (0, 0, 0)),
                  pl.BlockSpec((1, cout), lambda h: (0, 0))],
        out_specs=(pl.BlockSpec((N, 1, th, W, cout), lambda h: (0, h, 0, 0, 0)),
                   pl.BlockSpec((1, 1, cout), lambda h: (h, 0, 0)),
                   pl.BlockSpec((1, 1, cout), lambda h: (h, 0, 0))),
        scratch_shapes=[pltpu.VMEM((M, cout), jnp.float32)],
        compiler_params=pltpu.CompilerParams(
            dimension_semantics=("parallel",),
            vmem_limit_bytes=_VMEM_LIMIT),
    )(a, wm, bm)

    y = out.reshape(N, H, W, cout)
    s = jnp.sum(psum[:, 0, :], axis=0)
    ss = jnp.sum(psq[:, 0, :], axis=0)
    return y, (s, ss, N * H * W)


def _im2col(x, kh, kw, stride, pt, pb, pl_, pr):
    x = jnp.pad(x, ((0, 0), (pt, pb), (pl_, pr), (0, 0)))
    N, H, W, C = x.shape
    Ho = (H - kh) // stride + 1
    Wo = (W - kw) // stride + 1
    cols = []
    for i in range(kh):
        for j in range(kw):
            cols.append(x[:, i:i + (Ho - 1) * stride + 1:stride,
                            j:j + (Wo - 1) * stride + 1:stride, :])
    patches = jnp.stack(cols, axis=-2)
    return patches.reshape(N * Ho * Wo, kh * kw * C), (N, Ho, Wo)


def conv2d(x, p, stride=1, padding=(0, 0), out_dtype=jnp.bfloat16):
    w, b = p['w'], p['b']
    cout, cin, kh, kw = w.shape
    N, H, W, _ = x.shape
    if kh == 1 and kw == 1 and stride == 1:
        mat = x.reshape(N * H * W, cin)
        wmat = jnp.transpose(w, (2, 3, 1, 0)).reshape(cin, cout)
        out, s, ss = matmul_bias_stats(mat, wmat, b, out_dtype=out_dtype)
        return out.reshape(N, H, W, cout), (s, ss, N * H * W)
    if kh == 3 and kw == 3 and stride == 1 and padding == (1, 1) and H % 8 == 0:
        return conv3x3_direct(x, p, out_dtype=out_dtype)
    patches, (N, Ho, Wo) = _im2col(x.astype(jnp.bfloat16), kh, kw, stride,
                                   padding[0], padding[0], padding[1], padding[1])
    wmat = jnp.transpose(w, (2, 3, 1, 0)).reshape(kh * kw * cin, cout)
    out, s, ss = matmul_bias_stats(patches, wmat, b, out_dtype=out_dtype)
    return out.reshape(N, Ho, Wo, cout), (s, ss, N * Ho * Wo)


def _s2_taps(k, pad):
    taps = {0: [], 1: []}
    for d in (0, 1):
        for i in range(k):
            if (d + pad - i) % 2 == 0:
                taps[d].append(((d + pad - i) // 2, i))
    ts = sorted({t for d in (0, 1) for (t, _) in taps[d]})
    return taps, ts


def conv_transpose2d(x, p, padding=(0, 0), output_padding=(0, 0),
                     out_dtype=jnp.bfloat16):
    w, b = p['w'], p['b']
    cin, cout, kh, kw = w.shape
    N, H, W, _ = x.shape
    Ho = 2 * H
    Wo = 2 * W

    taps_h, ts_h = _s2_taps(kh, padding[0])
    taps_w, ts_w = _s2_taps(kw, padding[1])
    ph_lo, ph_hi = max(0, -ts_h[0]), max(0, ts_h[-1])
    pw_lo, pw_hi = max(0, -ts_w[0]), max(0, ts_w[-1])

    if len(ts_h) == 1 and len(ts_w) == 1 and ts_h[0] == 0 and ts_w[0] == 0:
        patches = x.astype(jnp.bfloat16).reshape(N * H * W, cin)
    else:
        xp = jnp.pad(x.astype(jnp.bfloat16),
                     ((0, 0), (ph_lo, ph_hi), (pw_lo, pw_hi), (0, 0)))
        cols = []
        for th in ts_h:
            for tw in ts_w:
                cols.append(xp[:, th + ph_lo: th + ph_lo + H,
                                 tw + pw_lo: tw + pw_lo + W, :])
        patches = jnp.concatenate(cols, axis=-1).reshape(
            N * H * W, len(ts_h) * len(ts_w) * cin)

    ih_tbl = np.full((len(ts_h), 2), -1, np.int64)
    for d in (0, 1):
        for (t, i) in taps_h[d]:
            ih_tbl[ts_h.index(t), d] = i
    iw_tbl = np.full((len(ts_w), 2), -1, np.int64)
    for d in (0, 1):
        for (t, i) in taps_w[d]:
            iw_tbl[ts_w.index(t), d] = i

    zeros = jnp.zeros((cin, cout), w.dtype)
    rows = []
    for a_ in range(len(ts_h)):
        for b_ in range(len(ts_w)):
            phase_cols = []
            for dh in (0, 1):
                for dw in (0, 1):
                    ih, iw = int(ih_tbl[a_, dh]), int(iw_tbl[b_, dw])
                    phase_cols.append(zeros if (ih < 0 or iw < 0) else w[:, :, ih, iw])
            rows.append(jnp.concatenate(phase_cols, axis=1))
    wmat = jnp.concatenate(rows, axis=0)

    out, s, ss = matmul_bias_stats(patches, wmat, jnp.tile(b, 4),
                                   out_dtype=out_dtype)
    out = out.reshape(N, H, W, 2, 2, cout)
    out = out.transpose(0, 1, 3, 2, 4, 5).reshape(N, 2 * H, 2 * W, cout)
    stats = (s.reshape(4, cout).sum(0), ss.reshape(4, cout).sum(0), N * Ho * Wo)
    return out, stats


def maxpool_3x3_s2_p1(x):
    xp = jnp.pad(x, ((0, 0), (1, 1), (1, 1), (0, 0)), constant_values=-jnp.inf)
    return lax.reduce_window(xp, -jnp.inf, lax.max, (1, 3, 3, 1), (1, 2, 2, 1),
                             'VALID')


def upsample_nearest(x, scale):
    if scale == 1:
        return x
    return jnp.repeat(jnp.repeat(x, scale, axis=1), scale, axis=2)


def conv_bn(x, cp, bnp, stride=1, padding=(0, 0), relu=True,
            residual=None, post_add=None):
    y, stats = conv2d(x, cp, stride=stride, padding=padding)
    return batchnorm2d(y, bnp, stats, relu=relu, residual=residual,
                       post_add=post_add)


def convT_bn(x, cp, bnp, padding=(0, 0), output_padding=(0, 0)):
    y, stats = conv_transpose2d(x, cp, padding=padding,
                                output_padding=output_padding)
    return batchnorm2d(y, bnp, stats, relu=True)


def basic_block(x, p, post_add=None):
    x = conv_bn(x, p['conv1'], p['bn1'], padding=(1, 1), relu=True)
    x = conv_bn(x, p['conv2'], p['bn2'], padding=(1, 1), relu=True,
                post_add=post_add)
    return x


def resnet_block(x, p, stride=1):
    identity = x
    out = conv_bn(x, p['conv1'], p['bn1'], stride=stride, padding=(1, 1),
                  relu=True)
    y, stats = conv2d(out, p['conv2'], stride=1, padding=(1, 1))
    if 'down_conv' in p:
        identity = conv_bn(x, p['down_conv'], p['down_bn'],
                           stride=stride, padding=(0, 0), relu=False)
    return batchnorm2d(y, p['bn2'], stats, relu=True, residual=identity)


def encoder(x, p):
    x = conv_bn(x, p['conv1'], p['bn1'], stride=2, padding=(3, 3), relu=True)
    x = maxpool_3x3_s2_p1(x)
    x = resnet_block(x, p['layer1'][0])
    x = resnet_block(x, p['layer1'][1])
    x1 = x
    x = resnet_block(x, p['layer2'][0], stride=2)
    x = resnet_block(x, p['layer2'][1])
    x2 = x
    x = resnet_block(x, p['layer3'][0], stride=2)
    x = resnet_block(x, p['layer3'][1])
    x3 = x
    x = resnet_block(x, p['layer4'][0], stride=2)
    x = resnet_block(x, p['layer4'][1])
    x4 = x
    return x1, x2, x3, x4


def upsample_module(x, p, padding=(0, 0), output_padding=(0, 0), skip=None):
    x = convT_bn(x, p['convT'], p['bn'], padding, output_padding)
    x = basic_block(x, p['conv'], post_add=skip)
    return x


def fpn_module(x, p, scale):
    x = conv_bn(x, p['conv1'], p['bn'], padding=(1, 1), relu=True)
    return upsample_nearest(x, scale)


def row_detection(x, p):
    x1, x2, x3, x4 = encoder(x, p['down'])
    x = upsample_module(x4, p['up1'], padding=(1, 1), output_padding=(1, 1),
                        skip=x3)
    x3 = x
    x = upsample_module(x, p['up2'], padding=(1, 1), output_padding=(1, 1),
                        skip=x2)
    x2 = x
    x = upsample_module(x, p['up3'], padding=(1, 1), output_padding=(1, 1),
                        skip=x1)
    x1 = x
    x = upsample_module(x, p['up4'], padding=(1, 1), output_padding=(1, 1))
    x3 = fpn_module(x3, p['fpn1'], 8)
    x2 = fpn_module(x2, p['fpn2'], 4)
    x1 = fpn_module(x1, p['fpn3'], 2)
    x = fpn_module(x, p['fpn4'], 1)
    x = jnp.concatenate([x, x1, x2, x3], axis=-1)
    x = upsample_module(x, p['up5'])
    x = basic_block(x, p['conv1'])
    y, _ = conv2d(x, p['conv2'], padding=(0, 0), out_dtype=jnp.float32)
    return y


class _ParamGen:
    def __init__(self, seed=0):
        self.key = jax.random.PRNGKey(seed)

    def next(self):
        self.key, sub = jax.random.split(self.key)
        return sub


def _make_conv(pg, cin, cout, k, bias=True):
    kh, kw = (k, k) if isinstance(k, int) else k
    w = jax.random.normal(pg.next(), (cout, cin, kh, kw), jnp.float32) / np.sqrt(cin * kh * kw)
    b = (jax.random.normal(pg.next(), (cout,), jnp.float32) * 0.01 if bias
         else jnp.zeros((cout,), jnp.float32))
    return {'w': w, 'b': b}


def _make_convT(pg, cin, cout, k):
    kh, kw = k
    w = jax.random.normal(pg.next(), (cin, cout, kh, kw), jnp.float32) / np.sqrt(cin * kh * kw)
    b = jax.random.normal(pg.next(), (cout,), jnp.float32) * 0.01
    return {'w': w, 'b': b}


def _make_bn(c):
    return {'gamma': jnp.ones((c,), jnp.float32), 'beta': jnp.zeros((c,), jnp.float32)}


def _make_resnet_block(pg, cin, cout, stride=1):
    p = {'conv1': _make_conv(pg, cin, cout, 3, bias=False), 'bn1': _make_bn(cout),
         'conv2': _make_conv(pg, cout, cout, 3, bias=False), 'bn2': _make_bn(cout)}
    if stride != 1 or cin != cout:
        p['down_conv'] = _make_conv(pg, cin, cout, 1, bias=False)
        p['down_bn'] = _make_bn(cout)
    return p


def _make_basic_block(pg, cin, cout):
    return {'conv1': _make_conv(pg, cin, cout, 3), 'bn1': _make_bn(cout),
            'conv2': _make_conv(pg, cout, cout, 3), 'bn2': _make_bn(cout)}


def _make_upsample(pg, cin, cout, k=(3, 3)):
    return {'convT': _make_convT(pg, cin, cout, k), 'bn': _make_bn(cout),
            'conv': _make_basic_block(pg, cout, cout)}


def _make_fpn(pg, cin):
    return {'conv1': _make_conv(pg, cin, 64, 3), 'bn': _make_bn(64)}


def _make_params(out_channel=2):
    pg = _ParamGen(0)
    down = {'conv1': _make_conv(pg, 3, 64, 7, bias=False), 'bn1': _make_bn(64),
            'layer1': [_make_resnet_block(pg, 64, 64), _make_resnet_block(pg, 64, 64)],
            'layer2': [_make_resnet_block(pg, 64, 128, 2), _make_resnet_block(pg, 128, 128)],
            'layer3': [_make_resnet_block(pg, 128, 256, 2), _make_resnet_block(pg, 256, 256)],
            'layer4': [_make_resnet_block(pg, 256, 512, 2), _make_resnet_block(pg, 512, 512)]}
    return {'down': down,
            'up1': _make_upsample(pg, 512, 256), 'up2': _make_upsample(pg, 256, 128),
            'up3': _make_upsample(pg, 128, 64), 'up4': _make_upsample(pg, 64, 64),
            'fpn1': _make_fpn(pg, 256), 'fpn2': _make_fpn(pg, 128),
            'fpn3': _make_fpn(pg, 64), 'fpn4': _make_fpn(pg, 64),
            'up5': _make_upsample(pg, 256, 64, k=(2, 2)),
            'conv1': _make_basic_block(pg, 64, 32),
            'conv2': _make_conv(pg, 32, out_channel, 1)}


_TREEDEF = None


def _treedef():
    global _TREEDEF
    if _TREEDEF is None:
        _, _TREEDEF = jax.tree_util.tree_flatten(_make_params(2))
    return _TREEDEF


def kernel(x_nchw, *leaves):
    params = jax.tree_util.tree_unflatten(_treedef(), list(leaves))
    x = jnp.transpose(x_nchw, (0, 2, 3, 1))
    y = row_detection(x, params)
    return jnp.transpose(y, (0, 3, 1, 2))


# speedup vs baseline: 2.4141x; 1.2147x over previous
import functools
import numpy as np
import jax
import jax.numpy as jnp
from jax import lax
from jax.experimental import pallas as pl
from jax.experimental.pallas import tpu as pltpu

_VMEM_LIMIT = 44 * 2**20


def _pick_tile(dim, cap, align):
    if dim <= cap:
        return dim
    t = (cap // align) * align
    while t >= align:
        if dim % t == 0:
            return t
        t -= align
    return dim


def _mm_kernel(a_ref, b_ref, bias_ref, o_ref, sum_ref, sq_ref):
    y = jnp.dot(a_ref[...], b_ref[...], preferred_element_type=jnp.float32)
    y = y + bias_ref[...]
    o_ref[...] = y.astype(o_ref.dtype)
    sum_ref[...] = jnp.sum(y, axis=0, keepdims=True)[None, :, :]
    sq_ref[...] = jnp.sum(y * y, axis=0, keepdims=True)[None, :, :]


def _mm_kernel_kt(a_ref, b_ref, bias_ref, o_ref, sum_ref, sq_ref, acc_ref):
    k = pl.program_id(2)

    @pl.when(k == 0)
    def _init():
        acc_ref[...] = jnp.zeros_like(acc_ref)

    acc_ref[...] += jnp.dot(a_ref[...], b_ref[...],
                            preferred_element_type=jnp.float32)

    @pl.when(k == pl.num_programs(2) - 1)
    def _store():
        y = acc_ref[...] + bias_ref[...]
        o_ref[...] = y.astype(o_ref.dtype)
        sum_ref[...] = jnp.sum(y, axis=0, keepdims=True)[None, :, :]
        sq_ref[...] = jnp.sum(y * y, axis=0, keepdims=True)[None, :, :]


def matmul_bias_stats(a, b, bias, out_dtype=jnp.bfloat16):
    M, K = a.shape
    Kb, N = b.shape
    assert K == Kb
    a = a.astype(jnp.bfloat16)
    b = b.astype(jnp.bfloat16)
    bias = bias.astype(jnp.float32)

    if K % 8 != 0:
        Kp = ((K + 127) // 128) * 128
        a = jnp.pad(a, ((0, 0), (0, Kp - K)))
        b = jnp.pad(b, ((0, Kp - K), (0, 0)))
        K = Kp
    n_orig = N
    if N % 8 != 0:
        Np = ((N + 127) // 128) * 128
        b = jnp.pad(b, ((0, 0), (0, Np - N)))
        bias = jnp.pad(bias, ((0, Np - N),))
        N = Np

    tm = _pick_tile(M, 2048, 16 if M % 16 == 0 else 8)
    tn = N if N <= 512 else _pick_tile(N, 512, 128)

    panel_bytes = 2 * (tm + tn) * K
    if panel_bytes <= 24 * 2**20:
        nmt, nnt = M // tm, N // tn
        out, psum, psq = pl.pallas_call(
            _mm_kernel,
            out_shape=(jax.ShapeDtypeStruct((M, N), out_dtype),
                       jax.ShapeDtypeStruct((nmt, 1, N), jnp.float32),
                       jax.ShapeDtypeStruct((nmt, 1, N), jnp.float32)),
            grid=(nmt, nnt),
            in_specs=[pl.BlockSpec((tm, K), lambda i, j: (i, 0)),
                      pl.BlockSpec((K, tn), lambda i, j: (0, j)),
                      pl.BlockSpec((1, tn), lambda i, j: (0, j))],
            out_specs=(pl.BlockSpec((tm, tn), lambda i, j: (i, j)),
                       pl.BlockSpec((1, 1, tn), lambda i, j: (i, 0, j)),
                       pl.BlockSpec((1, 1, tn), lambda i, j: (i, 0, j))),
            compiler_params=pltpu.CompilerParams(
                dimension_semantics=("parallel", "parallel"),
                vmem_limit_bytes=_VMEM_LIMIT),
        )(a, b, bias.reshape(1, N))
    else:
        tk = _pick_tile(K, 2048, 128)
        nmt, nnt, nkt = M // tm, N // tn, K // tk
        out, psum, psq = pl.pallas_call(
            _mm_kernel_kt,
            out_shape=(jax.ShapeDtypeStruct((M, N), out_dtype),
                       jax.ShapeDtypeStruct((nmt, 1, N), jnp.float32),
                       jax.ShapeDtypeStruct((nmt, 1, N), jnp.float32)),
            grid=(nmt, nnt, nkt),
            in_specs=[pl.BlockSpec((tm, tk), lambda i, j, k: (i, k)),
                      pl.BlockSpec((tk, tn), lambda i, j, k: (k, j)),
                      pl.BlockSpec((1, tn), lambda i, j, k: (0, j))],
            out_specs=(pl.BlockSpec((tm, tn), lambda i, j, k: (i, j)),
                       pl.BlockSpec((1, 1, tn), lambda i, j, k: (i, 0, j)),
                       pl.BlockSpec((1, 1, tn), lambda i, j, k: (i, 0, j))),
            scratch_shapes=[pltpu.VMEM((tm, tn), jnp.float32)],
            compiler_params=pltpu.CompilerParams(
                dimension_semantics=("parallel", "parallel", "arbitrary"),
                vmem_limit_bytes=_VMEM_LIMIT),
        )(a, b, bias.reshape(1, N))

    col_sum = jnp.sum(psum[:, 0, :], axis=0)
    col_sq = jnp.sum(psq[:, 0, :], axis=0)
    if n_orig != N:
        out = out[:, :n_orig]
        col_sum = col_sum[:n_orig]
        col_sq = col_sq[:n_orig]
    return out, col_sum, col_sq


def _bn_kernel(x_ref, scale_ref, shift_ref, o_ref, *, relu):
    y = x_ref[...].astype(jnp.float32) * scale_ref[...] + shift_ref[...]
    if relu:
        y = jnp.maximum(y, 0.0)
    o_ref[...] = y.astype(o_ref.dtype)


def _bn_res_kernel(x_ref, scale_ref, shift_ref, r_ref, o_ref, *, relu):
    y = (x_ref[...].astype(jnp.float32) * scale_ref[...] + shift_ref[...]
         + r_ref[...].astype(jnp.float32))
    if relu:
        y = jnp.maximum(y, 0.0)
    o_ref[...] = y.astype(o_ref.dtype)


def _bn_post_kernel(x_ref, scale_ref, shift_ref, r_ref, o_ref, *, relu):
    y = x_ref[...].astype(jnp.float32) * scale_ref[...] + shift_ref[...]
    if relu:
        y = jnp.maximum(y, 0.0)
    o_ref[...] = (y + r_ref[...].astype(jnp.float32)).astype(o_ref.dtype)


def _bn_scale_shift(p, stats, C, eps=1e-5):
    s, ss, cnt = stats
    mean = (s / cnt).reshape(1, C)
    var = jnp.maximum(ss / cnt - mean * mean, 0.0)
    scale = p['gamma'].reshape(1, C) * lax.rsqrt(var + eps)
    shift = p['beta'].reshape(1, C) - mean * scale
    return scale, shift


def batchnorm2d(x, p, stats, relu=False, residual=None, post_add=None,
                eps=1e-5, out_dtype=jnp.bfloat16):
    N, H, W, C = x.shape
    M = N * H * W
    scale, shift = _bn_scale_shift(p, stats, C, eps)

    extra = residual if residual is not None else post_add

    f = 1
    if C < 128 and 128 % C == 0 and M % (128 // C) == 0:
        f = 128 // C
    Mp, Cp = M // f, C * f
    x2 = x.reshape(Mp, Cp)
    scale_p = jnp.tile(scale, (1, f))
    shift_p = jnp.tile(shift, (1, f))

    tm = _pick_tile(Mp, 4096, 8)

    in_specs = [pl.BlockSpec((tm, Cp), lambda i: (i, 0)),
                pl.BlockSpec((1, Cp), lambda i: (0, 0)),
                pl.BlockSpec((1, Cp), lambda i: (0, 0))]
    args = [x2, scale_p, shift_p]
    if extra is not None:
        in_specs.append(pl.BlockSpec((tm, Cp), lambda i: (i, 0)))
        args.append(extra.reshape(Mp, Cp))
        kern = functools.partial(
            _bn_res_kernel if residual is not None else _bn_post_kernel, relu=relu)
    else:
        kern = functools.partial(_bn_kernel, relu=relu)

    out = pl.pallas_call(
        kern,
        out_shape=jax.ShapeDtypeStruct((Mp, Cp), out_dtype),
        grid=(Mp // tm,),
        in_specs=in_specs,
        out_specs=pl.BlockSpec((tm, Cp), lambda i: (i, 0)),
        compiler_params=pltpu.CompilerParams(
            dimension_semantics=("parallel",),
            vmem_limit_bytes=_VMEM_LIMIT),
    )(*args)
    return out.reshape(N, H, W, C)


def _dconv3_kernel(a_ref, w_ref, b_ref, o_ref, sum_ref, sq_ref, acc_ref, *, th):
    Nb = a_ref.shape[0]
    W = a_ref.shape[3]
    C3 = a_ref.shape[4]
    Co = o_ref.shape[4]
    for i in range(3):
        l = a_ref[:, 0, i:i + th, :, :].reshape(Nb * th * W, C3)
        d = jnp.dot(l, w_ref[i], preferred_element_type=jnp.float32)
        if i == 0:
            acc_ref[...] = d
        else:
            acc_ref[...] += d
    y = acc_ref[...] + b_ref[...]
    o_ref[...] = y.astype(o_ref.dtype).reshape(Nb, 1, th, W, Co)
    sum_ref[...] = jnp.sum(y, axis=0, keepdims=True)[None, :, :]
    sq_ref[...] = jnp.sum(y * y, axis=0, keepdims=True)[None, :, :]


def _dconv3b_kernel(x_ref, w_ref, b_ref, o_ref, sum_ref, sq_ref, p_ref, acc_ref,
                    *, th):
    Nb = x_ref.shape[0]
    W = o_ref.shape[3]
    C = x_ref.shape[4]
    Co = o_ref.shape[4]
    p_ref[...] = jnp.concatenate(
        [x_ref[:, 0, :, j:j + W, :] for j in range(3)], axis=-1)
    for i in range(3):
        l = p_ref[:, i:i + th].reshape(Nb * th * W, 3 * C)
        d = jnp.dot(l, w_ref[i], preferred_element_type=jnp.float32)
        if i == 0:
            acc_ref[...] = d
        else:
            acc_ref[...] += d
    y = acc_ref[...] + b_ref[...]
    o_ref[...] = y.astype(o_ref.dtype).reshape(Nb, 1, th, W, Co)
    sum_ref[...] = jnp.sum(y, axis=0, keepdims=True)[None, :, :]
    sq_ref[...] = jnp.sum(y * y, axis=0, keepdims=True)[None, :, :]


def conv3x3_direct(x, p, out_dtype=jnp.bfloat16):
    w, b = p['w'], p['b']
    cout, cin, _, _ = w.shape
    N, H, W, C = x.shape
    th = H if H <= 32 else 16
    HT = H // th

    xp = jnp.pad(x.astype(jnp.bfloat16), ((0, 0), (1, 1), (1, 1), (0, 0)))
    aw = jnp.concatenate([xp[:, :, 0:W, :], xp[:, :, 1:W + 1, :],
                          xp[:, :, 2:W + 2, :]], axis=-1)
    if HT == 1:
        a = aw[:, None]
    else:
        a = jnp.concatenate(
            [aw[:, h * th:h * th + th + 2][:, None] for h in range(HT)], axis=1)

    wm = jnp.transpose(w, (2, 3, 1, 0)).reshape(3, 3 * cin, cout).astype(jnp.bfloat16)
    bm = b.astype(jnp.float32).reshape(1, cout)
    M = N * th * W

    out, psum, psq = pl.pallas_call(
        functools.partial(_dconv3_kernel, th=th),
        out_shape=(jax.ShapeDtypeStruct((N, HT, th, W, cout), out_dtype),
                   jax.ShapeDtypeStruct((HT, 1, cout), jnp.float32),
                   jax.ShapeDtypeStruct((HT, 1, cout), jnp.float32)),
        grid=(HT,),
        in_specs=[pl.BlockSpec((N, 1, th + 2, W, 3 * C), lambda h: (0, h, 0, 0, 0)),
                  pl.BlockSpec((3, 3 * cin, cout), lambda h: (0, 0, 0)),
                  pl.BlockSpec((1, cout), lambda h: (0, 0))],
        out_specs=(pl.BlockSpec((N, 1, th, W, cout), lambda h: (0, h, 0, 0, 0)),
                   pl.BlockSpec((1, 1, cout), lambda h: (h, 0, 0)),
                   pl.BlockSpec((1, 1, cout), lambda h: (h, 0, 0))),
        scratch_shapes=[pltpu.VMEM((M, cout), jnp.float32)],
        compiler_params=pltpu.CompilerParams(
            dimension_semantics=("parallel",),
            vmem_limit_bytes=_VMEM_LIMIT),
    )(a, wm, bm)

    y = out.reshape(N, H, W, cout)
    s = jnp.sum(psum[:, 0, :], axis=0)
    ss = jnp.sum(psq[:, 0, :], axis=0)
    return y, (s, ss, N * H * W)


def conv3x3_direct_b(x, p, out_dtype=jnp.bfloat16):
    w, b = p['w'], p['b']
    cout, cin, _, _ = w.shape
    N, H, W, C = x.shape
    th = H if H <= 32 else 16
    HT = H // th

    xp = jnp.pad(x.astype(jnp.bfloat16), ((0, 0), (1, 1), (1, 1), (0, 0)))
    if HT == 1:
        a = xp[:, None]
    else:
        a = jnp.concatenate(
            [xp[:, h * th:h * th + th + 2][:, None] for h in range(HT)], axis=1)

    wm = jnp.transpose(w, (2, 3, 1, 0)).reshape(3, 3 * cin, cout).astype(jnp.bfloat16)
    bm = b.astype(jnp.float32).reshape(1, cout)
    M = N * th * W

    out, psum, psq = pl.pallas_call(
        functools.partial(_dconv3b_kernel, th=th),
        out_shape=(jax.ShapeDtypeStruct((N, HT, th, W, cout), out_dtype),
                   jax.ShapeDtypeStruct((HT, 1, cout), jnp.float32),
                   jax.ShapeDtypeStruct((HT, 1, cout), jnp.float32)),
        grid=(HT,),
        in_specs=[pl.BlockSpec((N, 1, th + 2, W + 2, C), lambda h: (0, h, 0, 0, 0)),
                  pl.BlockSpec((3, 3 * cin, cout), lambda h: (0, 0, 0)),
                  pl.BlockSpec((1, cout), lambda h: (0, 0))],
        out_specs=(pl.BlockSpec((N, 1, th, W, cout), lambda h: (0, h, 0, 0, 0)),
                   pl.BlockSpec((1, 1, cout), lambda h: (h, 0, 0)),
                   pl.BlockSpec((1, 1, cout), lambda h: (h, 0, 0))),
        scratch_shapes=[pltpu.VMEM((N, th + 2, W, 3 * C), jnp.bfloat16),
                        pltpu.VMEM((M, cout), jnp.float32)],
        compiler_params=pltpu.CompilerParams(
            dimension_semantics=("parallel",),
            vmem_limit_bytes=_VMEM_LIMIT),
    )(a, wm, bm)

    y = out.reshape(N, H, W, cout)
    s = jnp.sum(psum[:, 0, :], axis=0)
    ss = jnp.sum(psq[:, 0, :], axis=0)
    return y, (s, ss, N * H * W)


def _im2col(x, kh, kw, stride, pt, pb, pl_, pr):
    x = jnp.pad(x, ((0, 0), (pt, pb), (pl_, pr), (0, 0)))
    N, H, W, C = x.shape
    Ho = (H - kh) // stride + 1
    Wo = (W - kw) // stride + 1
    cols = []
    for i in range(kh):
        for j in range(kw):
            cols.append(x[:, i:i + (Ho - 1) * stride + 1:stride,
                            j:j + (Wo - 1) * stride + 1:stride, :])
    patches = jnp.stack(cols, axis=-2)
    return patches.reshape(N * Ho * Wo, kh * kw * C), (N, Ho, Wo)


def conv2d(x, p, stride=1, padding=(0, 0), out_dtype=jnp.bfloat16):
    w, b = p['w'], p['b']
    cout, cin, kh, kw = w.shape
    N, H, W, _ = x.shape
    if kh == 1 and kw == 1 and stride == 1:
        mat = x.reshape(N * H * W, cin)
        wmat = jnp.transpose(w, (2, 3, 1, 0)).reshape(cin, cout)
        out, s, ss = matmul_bias_stats(mat, wmat, b, out_dtype=out_dtype)
        return out.reshape(N, H, W, cout), (s, ss, N * H * W)
    if kh == 3 and kw == 3 and stride == 1 and padding == (1, 1) and H % 8 == 0:
        return conv3x3_direct_b(x, p, out_dtype=out_dtype)
    patches, (N, Ho, Wo) = _im2col(x.astype(jnp.bfloat16), kh, kw, stride,
                                   padding[0], padding[0], padding[1], padding[1])
    wmat = jnp.transpose(w, (2, 3, 1, 0)).reshape(kh * kw * cin, cout)
    out, s, ss = matmul_bias_stats(patches, wmat, b, out_dtype=out_dtype)
    return out.reshape(N, Ho, Wo, cout), (s, ss, N * Ho * Wo)


def _s2_taps(k, pad):
    taps = {0: [], 1: []}
    for d in (0, 1):
        for i in range(k):
            if (d + pad - i) % 2 == 0:
                taps[d].append(((d + pad - i) // 2, i))
    ts = sorted({t for d in (0, 1) for (t, _) in taps[d]})
    return taps, ts


def conv_transpose2d(x, p, padding=(0, 0), output_padding=(0, 0),
                     out_dtype=jnp.bfloat16):
    w, b = p['w'], p['b']
    cin, cout, kh, kw = w.shape
    N, H, W, _ = x.shape
    Ho = 2 * H
    Wo = 2 * W

    taps_h, ts_h = _s2_taps(kh, padding[0])
    taps_w, ts_w = _s2_taps(kw, padding[1])
    ph_lo, ph_hi = max(0, -ts_h[0]), max(0, ts_h[-1])
    pw_lo, pw_hi = max(0, -ts_w[0]), max(0, ts_w[-1])

    if len(ts_h) == 1 and len(ts_w) == 1 and ts_h[0] == 0 and ts_w[0] == 0:
        patches = x.astype(jnp.bfloat16).reshape(N * H * W, cin)
    else:
        xp = jnp.pad(x.astype(jnp.bfloat16),
                     ((0, 0), (ph_lo, ph_hi), (pw_lo, pw_hi), (0, 0)))
        cols = []
        for th in ts_h:
            for tw in ts_w:
                cols.append(xp[:, th + ph_lo: th + ph_lo + H,
                                 tw + pw_lo: tw + pw_lo + W, :])
        patches = jnp.concatenate(cols, axis=-1).reshape(
            N * H * W, len(ts_h) * len(ts_w) * cin)

    ih_tbl = np.full((len(ts_h), 2), -1, np.int64)
    for d in (0, 1):
        for (t, i) in taps_h[d]:
            ih_tbl[ts_h.index(t), d] = i
    iw_tbl = np.full((len(ts_w), 2), -1, np.int64)
    for d in (0, 1):
        for (t, i) in taps_w[d]:
            iw_tbl[ts_w.index(t), d] = i

    zeros = jnp.zeros((cin, cout), w.dtype)
    rows = []
    for a_ in range(len(ts_h)):
        for b_ in range(len(ts_w)):
            phase_cols = []
            for dh in (0, 1):
                for dw in (0, 1):
                    ih, iw = int(ih_tbl[a_, dh]), int(iw_tbl[b_, dw])
                    phase_cols.append(zeros if (ih < 0 or iw < 0) else w[:, :, ih, iw])
            rows.append(jnp.concatenate(phase_cols, axis=1))
    wmat = jnp.concatenate(rows, axis=0)

    out, s, ss = matmul_bias_stats(patches, wmat, jnp.tile(b, 4),
                                   out_dtype=out_dtype)
    out = out.reshape(N, H, W, 2, 2, cout)
    out = out.transpose(0, 1, 3, 2, 4, 5).reshape(N, 2 * H, 2 * W, cout)
    stats = (s.reshape(4, cout).sum(0), ss.reshape(4, cout).sum(0), N * Ho * Wo)
    return out, stats


def maxpool_3x3_s2_p1(x):
    xp = jnp.pad(x, ((0, 0), (1, 1), (1, 1), (0, 0)), constant_values=-jnp.inf)
    return lax.reduce_window(xp, -jnp.inf, lax.max, (1, 3, 3, 1), (1, 2, 2, 1),
                             'VALID')


def upsample_nearest(x, scale):
    if scale == 1:
        return x
    return jnp.repeat(jnp.repeat(x, scale, axis=1), scale, axis=2)


def conv_bn(x, cp, bnp, stride=1, padding=(0, 0), relu=True,
            residual=None, post_add=None):
    y, stats = conv2d(x, cp, stride=stride, padding=padding)
    return batchnorm2d(y, bnp, stats, relu=relu, residual=residual,
                       post_add=post_add)


def convT_bn(x, cp, bnp, padding=(0, 0), output_padding=(0, 0)):
    y, stats = conv_transpose2d(x, cp, padding=padding,
                                output_padding=output_padding)
    return batchnorm2d(y, bnp, stats, relu=True)


def basic_block(x, p, post_add=None):
    x = conv_bn(x, p['conv1'], p['bn1'], padding=(1, 1), relu=True)
    x = conv_bn(x, p['conv2'], p['bn2'], padding=(1, 1), relu=True,
                post_add=post_add)
    return x


def resnet_block(x, p, stride=1):
    identity = x
    out = conv_bn(x, p['conv1'], p['bn1'], stride=stride, padding=(1, 1),
                  relu=True)
    y, stats = conv2d(out, p['conv2'], stride=1, padding=(1, 1))
    if 'down_conv' in p:
        identity = conv_bn(x, p['down_conv'], p['down_bn'],
                           stride=stride, padding=(0, 0), relu=False)
    return batchnorm2d(y, p['bn2'], stats, relu=True, residual=identity)


def encoder(x, p):
    x = conv_bn(x, p['conv1'], p['bn1'], stride=2, padding=(3, 3), relu=True)
    x = maxpool_3x3_s2_p1(x)
    x = resnet_block(x, p['layer1'][0])
    x = resnet_block(x, p['layer1'][1])
    x1 = x
    x = resnet_block(x, p['layer2'][0], stride=2)
    x = resnet_block(x, p['layer2'][1])
    x2 = x
    x = resnet_block(x, p['layer3'][0], stride=2)
    x = resnet_block(x, p['layer3'][1])
    x3 = x
    x = resnet_block(x, p['layer4'][0], stride=2)
    x = resnet_block(x, p['layer4'][1])
    x4 = x
    return x1, x2, x3, x4


def upsample_module(x, p, padding=(0, 0), output_padding=(0, 0), skip=None):
    x = convT_bn(x, p['convT'], p['bn'], padding, output_padding)
    x = basic_block(x, p['conv'], post_add=skip)
    return x


def fpn_module(x, p, scale):
    x = conv_bn(x, p['conv1'], p['bn'], padding=(1, 1), relu=True)
    return upsample_nearest(x, scale)


def row_detection(x, p):
    x1, x2, x3, x4 = encoder(x, p['down'])
    x = upsample_module(x4, p['up1'], padding=(1, 1), output_padding=(1, 1),
                        skip=x3)
    x3 = x
    x = upsample_module(x, p['up2'], padding=(1, 1), output_padding=(1, 1),
                        skip=x2)
    x2 = x
    x = upsample_module(x, p['up3'], padding=(1, 1), output_padding=(1, 1),
                        skip=x1)
    x1 = x
    x = upsample_module(x, p['up4'], padding=(1, 1), output_padding=(1, 1))
    x3 = fpn_module(x3, p['fpn1'], 8)
    x2 = fpn_module(x2, p['fpn2'], 4)
    x1 = fpn_module(x1, p['fpn3'], 2)
    x = fpn_module(x, p['fpn4'], 1)
    x = jnp.concatenate([x, x1, x2, x3], axis=-1)
    x = upsample_module(x, p['up5'])
    x = basic_block(x, p['conv1'])
    y, _ = conv2d(x, p['conv2'], padding=(0, 0), out_dtype=jnp.float32)
    return y


class _ParamGen:
    def __init__(self, seed=0):
        self.key = jax.random.PRNGKey(seed)

    def next(self):
        self.key, sub = jax.random.split(self.key)
        return sub


def _make_conv(pg, cin, cout, k, bias=True):
    kh, kw = (k, k) if isinstance(k, int) else k
    w = jax.random.normal(pg.next(), (cout, cin, kh, kw), jnp.float32) / np.sqrt(cin * kh * kw)
    b = (jax.random.normal(pg.next(), (cout,), jnp.float32) * 0.01 if bias
         else jnp.zeros((cout,), jnp.float32))
    return {'w': w, 'b': b}


def _make_convT(pg, cin, cout, k):
    kh, kw = k
    w = jax.random.normal(pg.next(), (cin, cout, kh, kw), jnp.float32) / np.sqrt(cin * kh * kw)
    b = jax.random.normal(pg.next(), (cout,), jnp.float32) * 0.01
    return {'w': w, 'b': b}


def _make_bn(c):
    return {'gamma': jnp.ones((c,), jnp.float32), 'beta': jnp.zeros((c,), jnp.float32)}


def _make_resnet_block(pg, cin, cout, stride=1):
    p = {'conv1': _make_conv(pg, cin, cout, 3, bias=False), 'bn1': _make_bn(cout),
         'conv2': _make_conv(pg, cout, cout, 3, bias=False), 'bn2': _make_bn(cout)}
    if stride != 1 or cin != cout:
        p['down_conv'] = _make_conv(pg, cin, cout, 1, bias=False)
        p['down_bn'] = _make_bn(cout)
    return p


def _make_basic_block(pg, cin, cout):
    return {'conv1': _make_conv(pg, cin, cout, 3), 'bn1': _make_bn(cout),
            'conv2': _make_conv(pg, cout, cout, 3), 'bn2': _make_bn(cout)}


def _make_upsample(pg, cin, cout, k=(3, 3)):
    return {'convT': _make_convT(pg, cin, cout, k), 'bn': _make_bn(cout),
            'conv': _make_basic_block(pg, cout, cout)}


def _make_fpn(pg, cin):
    return {'conv1': _make_conv(pg, cin, 64, 3), 'bn': _make_bn(64)}


def _make_params(out_channel=2):
    pg = _ParamGen(0)
    down = {'conv1': _make_conv(pg, 3, 64, 7, bias=False), 'bn1': _make_bn(64),
            'layer1': [_make_resnet_block(pg, 64, 64), _make_resnet_block(pg, 64, 64)],
            'layer2': [_make_resnet_block(pg, 64, 128, 2), _make_resnet_block(pg, 128, 128)],
            'layer3': [_make_resnet_block(pg, 128, 256, 2), _make_resnet_block(pg, 256, 256)],
            'layer4': [_make_resnet_block(pg, 256, 512, 2), _make_resnet_block(pg, 512, 512)]}
    return {'down': down,
            'up1': _make_upsample(pg, 512, 256), 'up2': _make_upsample(pg, 256, 128),
            'up3': _make_upsample(pg, 128, 64), 'up4': _make_upsample(pg, 64, 64),
            'fpn1': _make_fpn(pg, 256), 'fpn2': _make_fpn(pg, 128),
            'fpn3': _make_fpn(pg, 64), 'fpn4': _make_fpn(pg, 64),
            'up5': _make_upsample(pg, 256, 64, k=(2, 2)),
            'conv1': _make_basic_block(pg, 64, 32),
            'conv2': _make_conv(pg, 32, out_channel, 1)}


_TREEDEF = None


def _treedef():
    global _TREEDEF
    if _TREEDEF is None:
        _, _TREEDEF = jax.tree_util.tree_flatten(_make_params(2))
    return _TREEDEF


def kernel(x_nchw, *leaves):
    params = jax.tree_util.tree_unflatten(_treedef(), list(leaves))
    x = jnp.transpose(x_nchw, (0, 2, 3, 1))
    y = row_detection(x, params)
    return jnp.transpose(y, (0, 3, 1, 2))


# speedup vs baseline: 2.4860x; 1.0298x over previous
import functools
import numpy as np
import jax
import jax.numpy as jnp
from jax import lax
from jax.experimental import pallas as pl
from jax.experimental.pallas import tpu as pltpu

_VMEM_LIMIT = 44 * 2**20


def _pick_tile(dim, cap, align):
    if dim <= cap:
        return dim
    t = (cap // align) * align
    while t >= align:
        if dim % t == 0:
            return t
        t -= align
    return dim


def _mm_kernel(a_ref, b_ref, bias_ref, o_ref, sum_ref, sq_ref):
    y = jnp.dot(a_ref[...], b_ref[...], preferred_element_type=jnp.float32)
    y = y + bias_ref[...]
    o_ref[...] = y.astype(o_ref.dtype)
    sum_ref[...] = jnp.sum(y, axis=0, keepdims=True)[None, :, :]
    sq_ref[...] = jnp.sum(y * y, axis=0, keepdims=True)[None, :, :]


def _mm_kernel_kt(a_ref, b_ref, bias_ref, o_ref, sum_ref, sq_ref, acc_ref):
    k = pl.program_id(2)

    @pl.when(k == 0)
    def _init():
        acc_ref[...] = jnp.zeros_like(acc_ref)

    acc_ref[...] += jnp.dot(a_ref[...], b_ref[...],
                            preferred_element_type=jnp.float32)

    @pl.when(k == pl.num_programs(2) - 1)
    def _store():
        y = acc_ref[...] + bias_ref[...]
        o_ref[...] = y.astype(o_ref.dtype)
        sum_ref[...] = jnp.sum(y, axis=0, keepdims=True)[None, :, :]
        sq_ref[...] = jnp.sum(y * y, axis=0, keepdims=True)[None, :, :]


def matmul_bias_stats(a, b, bias, out_dtype=jnp.bfloat16):
    M, K = a.shape
    Kb, N = b.shape
    assert K == Kb
    a = a.astype(jnp.bfloat16)
    b = b.astype(jnp.bfloat16)
    bias = bias.astype(jnp.float32)

    if K % 8 != 0:
        Kp = ((K + 127) // 128) * 128
        a = jnp.pad(a, ((0, 0), (0, Kp - K)))
        b = jnp.pad(b, ((0, Kp - K), (0, 0)))
        K = Kp
    n_orig = N
    if N % 8 != 0:
        Np = ((N + 127) // 128) * 128
        b = jnp.pad(b, ((0, 0), (0, Np - N)))
        bias = jnp.pad(bias, ((0, Np - N),))
        N = Np

    tm = _pick_tile(M, 2048, 16 if M % 16 == 0 else 8)
    tn = N if N <= 512 else _pick_tile(N, 512, 128)

    panel_bytes = 2 * (tm + tn) * K
    if panel_bytes <= 24 * 2**20:
        nmt, nnt = M // tm, N // tn
        out, psum, psq = pl.pallas_call(
            _mm_kernel,
            out_shape=(jax.ShapeDtypeStruct((M, N), out_dtype),
                       jax.ShapeDtypeStruct((nmt, 1, N), jnp.float32),
                       jax.ShapeDtypeStruct((nmt, 1, N), jnp.float32)),
            grid=(nmt, nnt),
            in_specs=[pl.BlockSpec((tm, K), lambda i, j: (i, 0)),
                      pl.BlockSpec((K, tn), lambda i, j: (0, j)),
                      pl.BlockSpec((1, tn), lambda i, j: (0, j))],
            out_specs=(pl.BlockSpec((tm, tn), lambda i, j: (i, j)),
                       pl.BlockSpec((1, 1, tn), lambda i, j: (i, 0, j)),
                       pl.BlockSpec((1, 1, tn), lambda i, j: (i, 0, j))),
            compiler_params=pltpu.CompilerParams(
                dimension_semantics=("parallel", "parallel"),
                vmem_limit_bytes=_VMEM_LIMIT),
        )(a, b, bias.reshape(1, N))
    else:
        tk = _pick_tile(K, 2048, 128)
        nmt, nnt, nkt = M // tm, N // tn, K // tk
        out, psum, psq = pl.pallas_call(
            _mm_kernel_kt,
            out_shape=(jax.ShapeDtypeStruct((M, N), out_dtype),
                       jax.ShapeDtypeStruct((nmt, 1, N), jnp.float32),
                       jax.ShapeDtypeStruct((nmt, 1, N), jnp.float32)),
            grid=(nmt, nnt, nkt),
            in_specs=[pl.BlockSpec((tm, tk), lambda i, j, k: (i, k)),
                      pl.BlockSpec((tk, tn), lambda i, j, k: (k, j)),
                      pl.BlockSpec((1, tn), lambda i, j, k: (0, j))],
            out_specs=(pl.BlockSpec((tm, tn), lambda i, j, k: (i, j)),
                       pl.BlockSpec((1, 1, tn), lambda i, j, k: (i, 0, j)),
                       pl.BlockSpec((1, 1, tn), lambda i, j, k: (i, 0, j))),
            scratch_shapes=[pltpu.VMEM((tm, tn), jnp.float32)],
            compiler_params=pltpu.CompilerParams(
                dimension_semantics=("parallel", "parallel", "arbitrary"),
                vmem_limit_bytes=_VMEM_LIMIT),
        )(a, b, bias.reshape(1, N))

    col_sum = jnp.sum(psum[:, 0, :], axis=0)
    col_sq = jnp.sum(psq[:, 0, :], axis=0)
    if n_orig != N:
        out = out[:, :n_orig]
        col_sum = col_sum[:n_orig]
        col_sq = col_sq[:n_orig]
    return out, col_sum, col_sq


def _bn_kernel(x_ref, scale_ref, shift_ref, o_ref, *, relu):
    y = x_ref[...].astype(jnp.float32) * scale_ref[...] + shift_ref[...]
    if relu:
        y = jnp.maximum(y, 0.0)
    o_ref[...] = y.astype(o_ref.dtype)


def _bn_res_kernel(x_ref, scale_ref, shift_ref, r_ref, o_ref, *, relu):
    y = (x_ref[...].astype(jnp.float32) * scale_ref[...] + shift_ref[...]
         + r_ref[...].astype(jnp.float32))
    if relu:
        y = jnp.maximum(y, 0.0)
    o_ref[...] = y.astype(o_ref.dtype)


def _bn_post_kernel(x_ref, scale_ref, shift_ref, r_ref, o_ref, *, relu):
    y = x_ref[...].astype(jnp.float32) * scale_ref[...] + shift_ref[...]
    if relu:
        y = jnp.maximum(y, 0.0)
    o_ref[...] = (y + r_ref[...].astype(jnp.float32)).astype(o_ref.dtype)


def _bn_scale_shift(p, stats, C, eps=1e-5):
    s, ss, cnt = stats
    mean = (s / cnt).reshape(1, C)
    var = jnp.maximum(ss / cnt - mean * mean, 0.0)
    scale = p['gamma'].reshape(1, C) * lax.rsqrt(var + eps)
    shift = p['beta'].reshape(1, C) - mean * scale
    return scale, shift


def batchnorm2d(x, p, stats, relu=False, residual=None, post_add=None,
                eps=1e-5, out_dtype=jnp.bfloat16):
    N, H, W, C = x.shape
    M = N * H * W
    scale, shift = _bn_scale_shift(p, stats, C, eps)

    extra = residual if residual is not None else post_add

    f = 1
    if C < 128 and 128 % C == 0 and M % (128 // C) == 0:
        f = 128 // C
    Mp, Cp = M // f, C * f
    x2 = x.reshape(Mp, Cp)
    scale_p = jnp.tile(scale, (1, f))
    shift_p = jnp.tile(shift, (1, f))

    tm = _pick_tile(Mp, 4096, 8)

    in_specs = [pl.BlockSpec((tm, Cp), lambda i: (i, 0)),
                pl.BlockSpec((1, Cp), lambda i: (0, 0)),
                pl.BlockSpec((1, Cp), lambda i: (0, 0))]
    args = [x2, scale_p, shift_p]
    if extra is not None:
        in_specs.append(pl.BlockSpec((tm, Cp), lambda i: (i, 0)))
        args.append(extra.reshape(Mp, Cp))
        kern = functools.partial(
            _bn_res_kernel if residual is not None else _bn_post_kernel, relu=relu)
    else:
        kern = functools.partial(_bn_kernel, relu=relu)

    out = pl.pallas_call(
        kern,
        out_shape=jax.ShapeDtypeStruct((Mp, Cp), out_dtype),
        grid=(Mp // tm,),
        in_specs=in_specs,
        out_specs=pl.BlockSpec((tm, Cp), lambda i: (i, 0)),
        compiler_params=pltpu.CompilerParams(
            dimension_semantics=("parallel",),
            vmem_limit_bytes=_VMEM_LIMIT),
    )(*args)
    return out.reshape(N, H, W, C)


def _dconv3_kernel(a_ref, w_ref, b_ref, o_ref, sum_ref, sq_ref, acc_ref, *, th):
    Nb = a_ref.shape[0]
    W = a_ref.shape[3]
    C3 = a_ref.shape[4]
    Co = o_ref.shape[4]
    for i in range(3):
        l = a_ref[:, 0, i:i + th, :, :].reshape(Nb * th * W, C3)
        d = jnp.dot(l, w_ref[i], preferred_element_type=jnp.float32)
        if i == 0:
            acc_ref[...] = d
        else:
            acc_ref[...] += d
    y = acc_ref[...] + b_ref[...]
    o_ref[...] = y.astype(o_ref.dtype).reshape(Nb, 1, th, W, Co)
    sum_ref[...] = jnp.sum(y, axis=0, keepdims=True)[None, :, :]
    sq_ref[...] = jnp.sum(y * y, axis=0, keepdims=True)[None, :, :]


def _dconv3b_kernel(x_ref, w_ref, b_ref, o_ref, sum_ref, sq_ref, p_ref, acc_ref,
                    *, th):
    Nb = x_ref.shape[0]
    W = o_ref.shape[3]
    C = x_ref.shape[4]
    Co = o_ref.shape[4]
    p_ref[...] = jnp.concatenate(
        [x_ref[:, 0, :, j:j + W, :] for j in range(3)], axis=-1)
    for i in range(3):
        l = p_ref[:, i:i + th].reshape(Nb * th * W, 3 * C)
        d = jnp.dot(l, w_ref[i], preferred_element_type=jnp.float32)
        if i == 0:
            acc_ref[...] = d
        else:
            acc_ref[...] += d
    y = acc_ref[...] + b_ref[...]
    o_ref[...] = y.astype(o_ref.dtype).reshape(Nb, 1, th, W, Co)
    sum_ref[...] = jnp.sum(y, axis=0, keepdims=True)[None, :, :]
    sq_ref[...] = jnp.sum(y * y, axis=0, keepdims=True)[None, :, :]


def conv3x3_direct(x, p, out_dtype=jnp.bfloat16):
    w, b = p['w'], p['b']
    cout, cin, _, _ = w.shape
    N, H, W, C = x.shape
    th = H if H <= 32 else 16
    HT = H // th

    xp = jnp.pad(x.astype(jnp.bfloat16), ((0, 0), (1, 1), (1, 1), (0, 0)))
    aw = jnp.concatenate([xp[:, :, 0:W, :], xp[:, :, 1:W + 1, :],
                          xp[:, :, 2:W + 2, :]], axis=-1)
    if HT == 1:
        a = aw[:, None]
    else:
        a = jnp.concatenate(
            [aw[:, h * th:h * th + th + 2][:, None] for h in range(HT)], axis=1)

    wm = jnp.transpose(w, (2, 3, 1, 0)).reshape(3, 3 * cin, cout).astype(jnp.bfloat16)
    bm = b.astype(jnp.float32).reshape(1, cout)
    M = N * th * W

    out, psum, psq = pl.pallas_call(
        functools.partial(_dconv3_kernel, th=th),
        out_shape=(jax.ShapeDtypeStruct((N, HT, th, W, cout), out_dtype),
                   jax.ShapeDtypeStruct((HT, 1, cout), jnp.float32),
                   jax.ShapeDtypeStruct((HT, 1, cout), jnp.float32)),
        grid=(HT,),
        in_specs=[pl.BlockSpec((N, 1, th + 2, W, 3 * C), lambda h: (0, h, 0, 0, 0)),
                  pl.BlockSpec((3, 3 * cin, cout), lambda h: (0, 0, 0)),
                  pl.BlockSpec((1, cout), lambda h: (0, 0))],
        out_specs=(pl.BlockSpec((N, 1, th, W, cout), lambda h: (0, h, 0, 0, 0)),
                   pl.BlockSpec((1, 1, cout), lambda h: (h, 0, 0)),
                   pl.BlockSpec((1, 1, cout), lambda h: (h, 0, 0))),
        scratch_shapes=[pltpu.VMEM((M, cout), jnp.float32)],
        compiler_params=pltpu.CompilerParams(
            dimension_semantics=("parallel",),
            vmem_limit_bytes=_VMEM_LIMIT),
    )(a, wm, bm)

    y = out.reshape(N, H, W, cout)
    s = jnp.sum(psum[:, 0, :], axis=0)
    ss = jnp.sum(psq[:, 0, :], axis=0)
    return y, (s, ss, N * H * W)


def _dconv3c_kernel(x_ref, w_ref, b_ref, o_ref, sum_ref, sq_ref, p_ref, acc_ref,
                    *, th):
    W = o_ref.shape[2]
    C = x_ref.shape[3]
    Co = o_ref.shape[3]
    h = pl.program_id(1)
    rows = x_ref[0, pl.ds(h * th, th + 2)]
    p_ref[...] = jnp.concatenate(
        [rows[:, j:j + W, :] for j in range(3)], axis=-1)
    for i in range(3):
        l = p_ref[i:i + th].reshape(th * W, 3 * C)
        d = jnp.dot(l, w_ref[i], preferred_element_type=jnp.float32)
        if i == 0:
            acc_ref[...] = d
        else:
            acc_ref[...] += d
    y = acc_ref[...] + b_ref[...]
    o_ref[...] = y.astype(o_ref.dtype).reshape(1, th, W, Co)
    sum_ref[...] = jnp.sum(y, axis=0, keepdims=True)[None, :, :]
    sq_ref[...] = jnp.sum(y * y, axis=0, keepdims=True)[None, :, :]


def conv3x3_direct_c(x, p, out_dtype=jnp.bfloat16):
    w, b = p['w'], p['b']
    cout, cin, _, _ = w.shape
    N, H, W, C = x.shape
    th = 16
    HT = H // th

    xp = jnp.pad(x.astype(jnp.bfloat16), ((0, 0), (1, 1), (1, 1), (0, 0)))
    wm = jnp.transpose(w, (2, 3, 1, 0)).reshape(3, 3 * cin, cout).astype(jnp.bfloat16)
    bm = b.astype(jnp.float32).reshape(1, cout)

    out, psum, psq = pl.pallas_call(
        functools.partial(_dconv3c_kernel, th=th),
        out_shape=(jax.ShapeDtypeStruct((N, H, W, cout), out_dtype),
                   jax.ShapeDtypeStruct((N * HT, 1, cout), jnp.float32),
                   jax.ShapeDtypeStruct((N * HT, 1, cout), jnp.float32)),
        grid=(N, HT),
        in_specs=[pl.BlockSpec((1, H + 2, W + 2, C), lambda n, h: (n, 0, 0, 0)),
                  pl.BlockSpec((3, 3 * cin, cout), lambda n, h: (0, 0, 0)),
                  pl.BlockSpec((1, cout), lambda n, h: (0, 0))],
        out_specs=(pl.BlockSpec((1, th, W, cout), lambda n, h: (n, h, 0, 0)),
                   pl.BlockSpec((1, 1, cout), lambda n, h: (n * HT + h, 0, 0)),
                   pl.BlockSpec((1, 1, cout), lambda n, h: (n * HT + h, 0, 0))),
        scratch_shapes=[pltpu.VMEM((th + 2, W, 3 * C), jnp.bfloat16),
                        pltpu.VMEM((th * W, cout), jnp.float32)],
        compiler_params=pltpu.CompilerParams(
            dimension_semantics=("parallel", "arbitrary"),
            vmem_limit_bytes=52 * 2**20),
    )(xp, wm, bm)

    s = jnp.sum(psum[:, 0, :], axis=0)
    ss = jnp.sum(psq[:, 0, :], axis=0)
    return out, (s, ss, N * H * W)


def conv3x3_direct_b(x, p, out_dtype=jnp.bfloat16):
    w, b = p['w'], p['b']
    cout, cin, _, _ = w.shape
    N, H, W, C = x.shape
    th = H if H <= 32 else 16
    HT = H // th

    xp = jnp.pad(x.astype(jnp.bfloat16), ((0, 0), (1, 1), (1, 1), (0, 0)))
    if HT == 1:
        a = xp[:, None]
    else:
        a = jnp.concatenate(
            [xp[:, h * th:h * th + th + 2][:, None] for h in range(HT)], axis=1)

    wm = jnp.transpose(w, (2, 3, 1, 0)).reshape(3, 3 * cin, cout).astype(jnp.bfloat16)
    bm = b.astype(jnp.float32).reshape(1, cout)
    M = N * th * W

    out, psum, psq = pl.pallas_call(
        functools.partial(_dconv3b_kernel, th=th),
        out_shape=(jax.ShapeDtypeStruct((N, HT, th, W, cout), out_dtype),
                   jax.ShapeDtypeStruct((HT, 1, cout), jnp.float32),
                   jax.ShapeDtypeStruct((HT, 1, cout), jnp.float32)),
        grid=(HT,),
        in_specs=[pl.BlockSpec((N, 1, th + 2, W + 2, C), lambda h: (0, h, 0, 0, 0)),
                  pl.BlockSpec((3, 3 * cin, cout), lambda h: (0, 0, 0)),
                  pl.BlockSpec((1, cout), lambda h: (0, 0))],
        out_specs=(pl.BlockSpec((N, 1, th, W, cout), lambda h: (0, h, 0, 0, 0)),
                   pl.BlockSpec((1, 1, cout), lambda h: (h, 0, 0)),
                   pl.BlockSpec((1, 1, cout), lambda h: (h, 0, 0))),
        scratch_shapes=[pltpu.VMEM((N, th + 2, W, 3 * C), jnp.bfloat16),
                        pltpu.VMEM((M, cout), jnp.float32)],
        compiler_params=pltpu.CompilerParams(
            dimension_semantics=("parallel",),
            vmem_limit_bytes=_VMEM_LIMIT),
    )(a, wm, bm)

    y = out.reshape(N, H, W, cout)
    s = jnp.sum(psum[:, 0, :], axis=0)
    ss = jnp.sum(psq[:, 0, :], axis=0)
    return y, (s, ss, N * H * W)


def _im2col(x, kh, kw, stride, pt, pb, pl_, pr):
    x = jnp.pad(x, ((0, 0), (pt, pb), (pl_, pr), (0, 0)))
    N, H, W, C = x.shape
    Ho = (H - kh) // stride + 1
    Wo = (W - kw) // stride + 1
    cols = []
    for i in range(kh):
        for j in range(kw):
            cols.append(x[:, i:i + (Ho - 1) * stride + 1:stride,
                            j:j + (Wo - 1) * stride + 1:stride, :])
    patches = jnp.stack(cols, axis=-2)
    return patches.reshape(N * Ho * Wo, kh * kw * C), (N, Ho, Wo)


def conv2d(x, p, stride=1, padding=(0, 0), out_dtype=jnp.bfloat16):
    w, b = p['w'], p['b']
    cout, cin, kh, kw = w.shape
    N, H, W, _ = x.shape
    if kh == 1 and kw == 1 and stride == 1:
        mat = x.reshape(N * H * W, cin)
        wmat = jnp.transpose(w, (2, 3, 1, 0)).reshape(cin, cout)
        out, s, ss = matmul_bias_stats(mat, wmat, b, out_dtype=out_dtype)
        return out.reshape(N, H, W, cout), (s, ss, N * H * W)
    if kh == 3 and kw == 3 and stride == 1 and padding == (1, 1) and H % 8 == 0:
        if H >= 64:
            return conv3x3_direct_c(x, p, out_dtype=out_dtype)
        return conv3x3_direct_b(x, p, out_dtype=out_dtype)
    patches, (N, Ho, Wo) = _im2col(x.astype(jnp.bfloat16), kh, kw, stride,
                                   padding[0], padding[0], padding[1], padding[1])
    wmat = jnp.transpose(w, (2, 3, 1, 0)).reshape(kh * kw * cin, cout)
    out, s, ss = matmul_bias_stats(patches, wmat, b, out_dtype=out_dtype)
    return out.reshape(N, Ho, Wo, cout), (s, ss, N * Ho * Wo)


def _s2_taps(k, pad):
    taps = {0: [], 1: []}
    for d in (0, 1):
        for i in range(k):
            if (d + pad - i) % 2 == 0:
                taps[d].append(((d + pad - i) // 2, i))
    ts = sorted({t for d in (0, 1) for (t, _) in taps[d]})
    return taps, ts


def conv_transpose2d(x, p, padding=(0, 0), output_padding=(0, 0),
                     out_dtype=jnp.bfloat16):
    w, b = p['w'], p['b']
    cin, cout, kh, kw = w.shape
    N, H, W, _ = x.shape
    Ho = 2 * H
    Wo = 2 * W

    taps_h, ts_h = _s2_taps(kh, padding[0])
    taps_w, ts_w = _s2_taps(kw, padding[1])
    ph_lo, ph_hi = max(0, -ts_h[0]), max(0, ts_h[-1])
    pw_lo, pw_hi = max(0, -ts_w[0]), max(0, ts_w[-1])

    if len(ts_h) == 1 and len(ts_w) == 1 and ts_h[0] == 0 and ts_w[0] == 0:
        patches = x.astype(jnp.bfloat16).reshape(N * H * W, cin)
    else:
        xp = jnp.pad(x.astype(jnp.bfloat16),
                     ((0, 0), (ph_lo, ph_hi), (pw_lo, pw_hi), (0, 0)))
        cols = []
        for th in ts_h:
            for tw in ts_w:
                cols.append(xp[:, th + ph_lo: th + ph_lo + H,
                                 tw + pw_lo: tw + pw_lo + W, :])
        patches = jnp.concatenate(cols, axis=-1).reshape(
            N * H * W, len(ts_h) * len(ts_w) * cin)

    ih_tbl = np.full((len(ts_h), 2), -1, np.int64)
    for d in (0, 1):
        for (t, i) in taps_h[d]:
            ih_tbl[ts_h.index(t), d] = i
    iw_tbl = np.full((len(ts_w), 2), -1, np.int64)
    for d in (0, 1):
        for (t, i) in taps_w[d]:
            iw_tbl[ts_w.index(t), d] = i

    zeros = jnp.zeros((cin, cout), w.dtype)
    rows = []
    for a_ in range(len(ts_h)):
        for b_ in range(len(ts_w)):
            phase_cols = []
            for dh in (0, 1):
                for dw in (0, 1):
                    ih, iw = int(ih_tbl[a_, dh]), int(iw_tbl[b_, dw])
                    phase_cols.append(zeros if (ih < 0 or iw < 0) else w[:, :, ih, iw])
            rows.append(jnp.concatenate(phase_cols, axis=1))
    wmat = jnp.concatenate(rows, axis=0)

    out, s, ss = matmul_bias_stats(patches, wmat, jnp.tile(b, 4),
                                   out_dtype=out_dtype)
    out = out.reshape(N, H, W, 2, 2, cout)
    out = out.transpose(0, 1, 3, 2, 4, 5).reshape(N, 2 * H, 2 * W, cout)
    stats = (s.reshape(4, cout).sum(0), ss.reshape(4, cout).sum(0), N * Ho * Wo)
    return out, stats


def maxpool_3x3_s2_p1(x):
    xp = jnp.pad(x, ((0, 0), (1, 1), (1, 1), (0, 0)), constant_values=-jnp.inf)
    return lax.reduce_window(xp, -jnp.inf, lax.max, (1, 3, 3, 1), (1, 2, 2, 1),
                             'VALID')


def upsample_nearest(x, scale):
    if scale == 1:
        return x
    return jnp.repeat(jnp.repeat(x, scale, axis=1), scale, axis=2)


def conv_bn(x, cp, bnp, stride=1, padding=(0, 0), relu=True,
            residual=None, post_add=None):
    y, stats = conv2d(x, cp, stride=stride, padding=padding)
    return batchnorm2d(y, bnp, stats, relu=relu, residual=residual,
                       post_add=post_add)


def convT_bn(x, cp, bnp, padding=(0, 0), output_padding=(0, 0)):
    y, stats = conv_transpose2d(x, cp, padding=padding,
                                output_padding=output_padding)
    return batchnorm2d(y, bnp, stats, relu=True)


def basic_block(x, p, post_add=None):
    x = conv_bn(x, p['conv1'], p['bn1'], padding=(1, 1), relu=True)
    x = conv_bn(x, p['conv2'], p['bn2'], padding=(1, 1), relu=True,
                post_add=post_add)
    return x


def resnet_block(x, p, stride=1):
    identity = x
    out = conv_bn(x, p['conv1'], p['bn1'], stride=stride, padding=(1, 1),
                  relu=True)
    y, stats = conv2d(out, p['conv2'], stride=1, padding=(1, 1))
    if 'down_conv' in p:
        identity = conv_bn(x, p['down_conv'], p['down_bn'],
                           stride=stride, padding=(0, 0), relu=False)
    return batchnorm2d(y, p['bn2'], stats, relu=True, residual=identity)


def encoder(x, p):
    x = conv_bn(x, p['conv1'], p['bn1'], stride=2, padding=(3, 3), relu=True)
    x = maxpool_3x3_s2_p1(x)
    x = resnet_block(x, p['layer1'][0])
    x = resnet_block(x, p['layer1'][1])
    x1 = x
    x = resnet_block(x, p['layer2'][0], stride=2)
    x = resnet_block(x, p['layer2'][1])
    x2 = x
    x = resnet_block(x, p['layer3'][0], stride=2)
    x = resnet_block(x, p['layer3'][1])
    x3 = x
    x = resnet_block(x, p['layer4'][0], stride=2)
    x = resnet_block(x, p['layer4'][1])
    x4 = x
    return x1, x2, x3, x4


def upsample_module(x, p, padding=(0, 0), output_padding=(0, 0), skip=None):
    x = convT_bn(x, p['convT'], p['bn'], padding, output_padding)
    x = basic_block(x, p['conv'], post_add=skip)
    return x


def fpn_module(x, p, scale):
    x = conv_bn(x, p['conv1'], p['bn'], padding=(1, 1), relu=True)
    return upsample_nearest(x, scale)


def row_detection(x, p):
    x1, x2, x3, x4 = encoder(x, p['down'])
    x = upsample_module(x4, p['up1'], padding=(1, 1), output_padding=(1, 1),
                        skip=x3)
    x3 = x
    x = upsample_module(x, p['up2'], padding=(1, 1), output_padding=(1, 1),
                        skip=x2)
    x2 = x
    x = upsample_module(x, p['up3'], padding=(1, 1), output_padding=(1, 1),
                        skip=x1)
    x1 = x
    x = upsample_module(x, p['up4'], padding=(1, 1), output_padding=(1, 1))
    x3 = fpn_module(x3, p['fpn1'], 8)
    x2 = fpn_module(x2, p['fpn2'], 4)
    x1 = fpn_module(x1, p['fpn3'], 2)
    x = fpn_module(x, p['fpn4'], 1)
    x = jnp.concatenate([x, x1, x2, x3], axis=-1)
    x = upsample_module(x, p['up5'])
    x = basic_block(x, p['conv1'])
    y, _ = conv2d(x, p['conv2'], padding=(0, 0), out_dtype=jnp.bfloat16)
    return y.astype(jnp.float32)


class _ParamGen:
    def __init__(self, seed=0):
        self.key = jax.random.PRNGKey(seed)

    def next(self):
        self.key, sub = jax.random.split(self.key)
        return sub


def _make_conv(pg, cin, cout, k, bias=True):
    kh, kw = (k, k) if isinstance(k, int) else k
    w = jax.random.normal(pg.next(), (cout, cin, kh, kw), jnp.float32) / np.sqrt(cin * kh * kw)
    b = (jax.random.normal(pg.next(), (cout,), jnp.float32) * 0.01 if bias
         else jnp.zeros((cout,), jnp.float32))
    return {'w': w, 'b': b}


def _make_convT(pg, cin, cout, k):
    kh, kw = k
    w = jax.random.normal(pg.next(), (cin, cout, kh, kw), jnp.float32) / np.sqrt(cin * kh * kw)
    b = jax.random.normal(pg.next(), (cout,), jnp.float32) * 0.01
    return {'w': w, 'b': b}


def _make_bn(c):
    return {'gamma': jnp.ones((c,), jnp.float32), 'beta': jnp.zeros((c,), jnp.float32)}


def _make_resnet_block(pg, cin, cout, stride=1):
    p = {'conv1': _make_conv(pg, cin, cout, 3, bias=False), 'bn1': _make_bn(cout),
         'conv2': _make_conv(pg, cout, cout, 3, bias=False), 'bn2': _make_bn(cout)}
    if stride != 1 or cin != cout:
        p['down_conv'] = _make_conv(pg, cin, cout, 1, bias=False)
        p['down_bn'] = _make_bn(cout)
    return p


def _make_basic_block(pg, cin, cout):
    return {'conv1': _make_conv(pg, cin, cout, 3), 'bn1': _make_bn(cout),
            'conv2': _make_conv(pg, cout, cout, 3), 'bn2': _make_bn(cout)}


def _make_upsample(pg, cin, cout, k=(3, 3)):
    return {'convT': _make_convT(pg, cin, cout, k), 'bn': _make_bn(cout),
            'conv': _make_basic_block(pg, cout, cout)}


def _make_fpn(pg, cin):
    return {'conv1': _make_conv(pg, cin, 64, 3), 'bn': _make_bn(64)}


def _make_params(out_channel=2):
    pg = _ParamGen(0)
    down = {'conv1': _make_conv(pg, 3, 64, 7, bias=False), 'bn1': _make_bn(64),
            'layer1': [_make_resnet_block(pg, 64, 64), _make_resnet_block(pg, 64, 64)],
            'layer2': [_make_resnet_block(pg, 64, 128, 2), _make_resnet_block(pg, 128, 128)],
            'layer3': [_make_resnet_block(pg, 128, 256, 2), _make_resnet_block(pg, 256, 256)],
            'layer4': [_make_resnet_block(pg, 256, 512, 2), _make_resnet_block(pg, 512, 512)]}
    return {'down': down,
            'up1': _make_upsample(pg, 512, 256), 'up2': _make_upsample(pg, 256, 128),
            'up3': _make_upsample(pg, 128, 64), 'up4': _make_upsample(pg, 64, 64),
            'fpn1': _make_fpn(pg, 256), 'fpn2': _make_fpn(pg, 128),
            'fpn3': _make_fpn(pg, 64), 'fpn4': _make_fpn(pg, 64),
            'up5': _make_upsample(pg, 256, 64, k=(2, 2)),
            'conv1': _make_basic_block(pg, 64, 32),
            'conv2': _make_conv(pg, 32, out_channel, 1)}


_TREEDEF = None


def _treedef():
    global _TREEDEF
    if _TREEDEF is None:
        _, _TREEDEF = jax.tree_util.tree_flatten(_make_params(2))
    return _TREEDEF


def kernel(x_nchw, *leaves):
    params = jax.tree_util.tree_unflatten(_treedef(), list(leaves))
    x = jnp.transpose(x_nchw, (0, 2, 3, 1))
    y = row_detection(x, params)
    return jnp.transpose(y, (0, 3, 1, 2))


# speedup vs baseline: 4.1531x; 1.6706x over previous
import functools
import numpy as np
import jax
import jax.numpy as jnp
from jax import lax
from jax.experimental import pallas as pl
from jax.experimental.pallas import tpu as pltpu

_VMEM_LIMIT = 44 * 2**20


def _pick_tile(dim, cap, align):
    if dim <= cap:
        return dim
    t = (cap // align) * align
    while t >= align:
        if dim % t == 0:
            return t
        t -= align
    return dim


def _mm_kernel(a_ref, b_ref, bias_ref, o_ref, sum_ref, sq_ref):
    y = jnp.dot(a_ref[...], b_ref[...], preferred_element_type=jnp.float32)
    y = y + bias_ref[...]
    o_ref[...] = y.astype(o_ref.dtype)
    sum_ref[...] = jnp.sum(y, axis=0, keepdims=True)[None, :, :]
    sq_ref[...] = jnp.sum(y * y, axis=0, keepdims=True)[None, :, :]


def _mm_kernel_kt(a_ref, b_ref, bias_ref, o_ref, sum_ref, sq_ref, acc_ref):
    k = pl.program_id(2)

    @pl.when(k == 0)
    def _init():
        acc_ref[...] = jnp.zeros_like(acc_ref)

    acc_ref[...] += jnp.dot(a_ref[...], b_ref[...],
                            preferred_element_type=jnp.float32)

    @pl.when(k == pl.num_programs(2) - 1)
    def _store():
        y = acc_ref[...] + bias_ref[...]
        o_ref[...] = y.astype(o_ref.dtype)
        sum_ref[...] = jnp.sum(y, axis=0, keepdims=True)[None, :, :]
        sq_ref[...] = jnp.sum(y * y, axis=0, keepdims=True)[None, :, :]


def matmul_bias_stats(a, b, bias, out_dtype=jnp.bfloat16):
    M, K = a.shape
    Kb, N = b.shape
    assert K == Kb
    a = a.astype(jnp.bfloat16)
    b = b.astype(jnp.bfloat16)
    bias = bias.astype(jnp.float32)

    if K % 8 != 0:
        Kp = ((K + 127) // 128) * 128
        a = jnp.pad(a, ((0, 0), (0, Kp - K)))
        b = jnp.pad(b, ((0, Kp - K), (0, 0)))
        K = Kp
    n_orig = N
    if N % 8 != 0:
        Np = ((N + 127) // 128) * 128
        b = jnp.pad(b, ((0, 0), (0, Np - N)))
        bias = jnp.pad(bias, ((0, Np - N),))
        N = Np

    tm = _pick_tile(M, 2048, 16 if M % 16 == 0 else 8)
    tn = N if N <= 512 else _pick_tile(N, 512, 128)

    panel_bytes = 2 * (tm + tn) * K
    if panel_bytes <= 24 * 2**20:
        nmt, nnt = M // tm, N // tn
        out, psum, psq = pl.pallas_call(
            _mm_kernel,
            out_shape=(jax.ShapeDtypeStruct((M, N), out_dtype),
                       jax.ShapeDtypeStruct((nmt, 1, N), jnp.float32),
                       jax.ShapeDtypeStruct((nmt, 1, N), jnp.float32)),
            grid=(nmt, nnt),
            in_specs=[pl.BlockSpec((tm, K), lambda i, j: (i, 0)),
                      pl.BlockSpec((K, tn), lambda i, j: (0, j)),
                      pl.BlockSpec((1, tn), lambda i, j: (0, j))],
            out_specs=(pl.BlockSpec((tm, tn), lambda i, j: (i, j)),
                       pl.BlockSpec((1, 1, tn), lambda i, j: (i, 0, j)),
                       pl.BlockSpec((1, 1, tn), lambda i, j: (i, 0, j))),
            compiler_params=pltpu.CompilerParams(
                dimension_semantics=("parallel", "parallel"),
                vmem_limit_bytes=_VMEM_LIMIT),
        )(a, b, bias.reshape(1, N))
    else:
        tk = _pick_tile(K, 2048, 128)
        nmt, nnt, nkt = M // tm, N // tn, K // tk
        out, psum, psq = pl.pallas_call(
            _mm_kernel_kt,
            out_shape=(jax.ShapeDtypeStruct((M, N), out_dtype),
                       jax.ShapeDtypeStruct((nmt, 1, N), jnp.float32),
                       jax.ShapeDtypeStruct((nmt, 1, N), jnp.float32)),
            grid=(nmt, nnt, nkt),
            in_specs=[pl.BlockSpec((tm, tk), lambda i, j, k: (i, k)),
                      pl.BlockSpec((tk, tn), lambda i, j, k: (k, j)),
                      pl.BlockSpec((1, tn), lambda i, j, k: (0, j))],
            out_specs=(pl.BlockSpec((tm, tn), lambda i, j, k: (i, j)),
                       pl.BlockSpec((1, 1, tn), lambda i, j, k: (i, 0, j)),
                       pl.BlockSpec((1, 1, tn), lambda i, j, k: (i, 0, j))),
            scratch_shapes=[pltpu.VMEM((tm, tn), jnp.float32)],
            compiler_params=pltpu.CompilerParams(
                dimension_semantics=("parallel", "parallel", "arbitrary"),
                vmem_limit_bytes=_VMEM_LIMIT),
        )(a, b, bias.reshape(1, N))

    col_sum = jnp.sum(psum[:, 0, :], axis=0)
    col_sq = jnp.sum(psq[:, 0, :], axis=0)
    if n_orig != N:
        out = out[:, :n_orig]
        col_sum = col_sum[:n_orig]
        col_sq = col_sq[:n_orig]
    return out, col_sum, col_sq


def _bn_kernel(x_ref, scale_ref, shift_ref, o_ref, *, relu):
    y = x_ref[...].astype(jnp.float32) * scale_ref[...] + shift_ref[...]
    if relu:
        y = jnp.maximum(y, 0.0)
    o_ref[...] = y.astype(o_ref.dtype)


def _bn_res_kernel(x_ref, scale_ref, shift_ref, r_ref, o_ref, *, relu):
    y = (x_ref[...].astype(jnp.float32) * scale_ref[...] + shift_ref[...]
         + r_ref[...].astype(jnp.float32))
    if relu:
        y = jnp.maximum(y, 0.0)
    o_ref[...] = y.astype(o_ref.dtype)


def _bn_post_kernel(x_ref, scale_ref, shift_ref, r_ref, o_ref, *, relu):
    y = x_ref[...].astype(jnp.float32) * scale_ref[...] + shift_ref[...]
    if relu:
        y = jnp.maximum(y, 0.0)
    o_ref[...] = (y + r_ref[...].astype(jnp.float32)).astype(o_ref.dtype)


def _bn_scale_shift(p, stats, C, eps=1e-5):
    s, ss, cnt = stats
    mean = (s / cnt).reshape(1, C)
    var = jnp.maximum(ss / cnt - mean * mean, 0.0)
    scale = p['gamma'].reshape(1, C) * lax.rsqrt(var + eps)
    shift = p['beta'].reshape(1, C) - mean * scale
    return scale, shift


def batchnorm2d(x, p, stats, relu=False, residual=None, post_add=None,
                eps=1e-5, out_dtype=jnp.bfloat16):
    N, H, W, C = x.shape
    M = N * H * W
    scale, shift = _bn_scale_shift(p, stats, C, eps)

    extra = residual if residual is not None else post_add

    Mp, Cp = M, C
    x2 = x.reshape(Mp, Cp)
    scale_p = scale
    shift_p = shift

    tm = _pick_tile(Mp, 4096, 8)

    in_specs = [pl.BlockSpec((tm, Cp), lambda i: (i, 0)),
                pl.BlockSpec((1, Cp), lambda i: (0, 0)),
                pl.BlockSpec((1, Cp), lambda i: (0, 0))]
    args = [x2, scale_p, shift_p]
    if extra is not None:
        in_specs.append(pl.BlockSpec((tm, Cp), lambda i: (i, 0)))
        args.append(extra.reshape(Mp, Cp))
        kern = functools.partial(
            _bn_res_kernel if residual is not None else _bn_post_kernel, relu=relu)
    else:
        kern = functools.partial(_bn_kernel, relu=relu)

    out = pl.pallas_call(
        kern,
        out_shape=jax.ShapeDtypeStruct((Mp, Cp), out_dtype),
        grid=(Mp // tm,),
        in_specs=in_specs,
        out_specs=pl.BlockSpec((tm, Cp), lambda i: (i, 0)),
        compiler_params=pltpu.CompilerParams(
            dimension_semantics=("parallel",),
            vmem_limit_bytes=_VMEM_LIMIT),
    )(*args)
    return out.reshape(N, H, W, C)


def _dconv3b_kernel(x_ref, w_ref, b_ref, o_ref, sum_ref, sq_ref, p_ref, acc_ref,
                    *, th):
    Nb = x_ref.shape[0]
    W = o_ref.shape[3]
    C = x_ref.shape[4]
    Co = o_ref.shape[4]
    p_ref[...] = jnp.concatenate(
        [x_ref[:, 0, :, j:j + W, :] for j in range(3)], axis=-1)
    for i in range(3):
        l = p_ref[:, i:i + th].reshape(Nb * th * W, 3 * C)
        d = jnp.dot(l, w_ref[i], preferred_element_type=jnp.float32)
        if i == 0:
            acc_ref[...] = d
        else:
            acc_ref[...] += d
    y = acc_ref[...] + b_ref[...]
    o_ref[...] = y.astype(o_ref.dtype).reshape(Nb, 1, th, W, Co)
    sum_ref[...] = jnp.sum(y, axis=0, keepdims=True)[None, :, :]
    sq_ref[...] = jnp.sum(y * y, axis=0, keepdims=True)[None, :, :]


def _dconv3c_kernel(x_ref, w_ref, b_ref, o_ref, sum_ref, sq_ref, p_ref, acc_ref,
                    *, th):
    W = o_ref.shape[2]
    C = x_ref.shape[3]
    Co = o_ref.shape[3]
    h = pl.program_id(1)
    rows = x_ref[0, pl.ds(h * th, th + 2)]
    p_ref[...] = jnp.concatenate(
        [rows[:, j:j + W, :] for j in range(3)], axis=-1)
    for i in range(3):
        l = p_ref[i:i + th].reshape(th * W, 3 * C)
        d = jnp.dot(l, w_ref[i], preferred_element_type=jnp.float32)
        if i == 0:
            acc_ref[...] = d
        else:
            acc_ref[...] += d
    y = acc_ref[...] + b_ref[...]
    o_ref[...] = y.astype(o_ref.dtype).reshape(1, th, W, Co)
    sum_ref[...] = jnp.sum(y, axis=0, keepdims=True)[None, :, :]
    sq_ref[...] = jnp.sum(y * y, axis=0, keepdims=True)[None, :, :]


def conv3x3_direct_c(x, p, out_dtype=jnp.bfloat16):
    w, b = p['w'], p['b']
    cout, cin, _, _ = w.shape
    N, H, W, C = x.shape
    th = 32 if H % 32 == 0 else 16
    HT = H // th

    xp = jnp.pad(x.astype(jnp.bfloat16), ((0, 0), (1, 1), (1, 1), (0, 0)))
    wm = jnp.transpose(w, (2, 3, 1, 0)).reshape(3, 3 * cin, cout).astype(jnp.bfloat16)
    bm = b.astype(jnp.float32).reshape(1, cout)

    out, psum, psq = pl.pallas_call(
        functools.partial(_dconv3c_kernel, th=th),
        out_shape=(jax.ShapeDtypeStruct((N, H, W, cout), out_dtype),
                   jax.ShapeDtypeStruct((N * HT, 1, cout), jnp.float32),
                   jax.ShapeDtypeStruct((N * HT, 1, cout), jnp.float32)),
        grid=(N, HT),
        in_specs=[pl.BlockSpec((1, H + 2, W + 2, C), lambda n, h: (n, 0, 0, 0)),
                  pl.BlockSpec((3, 3 * cin, cout), lambda n, h: (0, 0, 0)),
                  pl.BlockSpec((1, cout), lambda n, h: (0, 0))],
        out_specs=(pl.BlockSpec((1, th, W, cout), lambda n, h: (n, h, 0, 0)),
                   pl.BlockSpec((1, 1, cout), lambda n, h: (n * HT + h, 0, 0)),
                   pl.BlockSpec((1, 1, cout), lambda n, h: (n * HT + h, 0, 0))),
        scratch_shapes=[pltpu.VMEM((th + 2, W, 3 * C), jnp.bfloat16),
                        pltpu.VMEM((th * W, cout), jnp.float32)],
        compiler_params=pltpu.CompilerParams(
            dimension_semantics=("parallel", "arbitrary"),
            vmem_limit_bytes=52 * 2**20),
    )(xp, wm, bm)

    s = jnp.sum(psum[:, 0, :], axis=0)
    ss = jnp.sum(psq[:, 0, :], axis=0)
    return out, (s, ss, N * H * W)


def conv3x3_direct_b(x, p, out_dtype=jnp.bfloat16):
    w, b = p['w'], p['b']
    cout, cin, _, _ = w.shape
    N, H, W, C = x.shape
    th = H if H <= 32 else 16
    HT = H // th

    xp = jnp.pad(x.astype(jnp.bfloat16), ((0, 0), (1, 1), (1, 1), (0, 0)))
    if HT == 1:
        a = xp[:, None]
    else:
        a = jnp.concatenate(
            [xp[:, h * th:h * th + th + 2][:, None] for h in range(HT)], axis=1)

    wm = jnp.transpose(w, (2, 3, 1, 0)).reshape(3, 3 * cin, cout).astype(jnp.bfloat16)
    bm = b.astype(jnp.float32).reshape(1, cout)
    M = N * th * W

    out, psum, psq = pl.pallas_call(
        functools.partial(_dconv3b_kernel, th=th),
        out_shape=(jax.ShapeDtypeStruct((N, HT, th, W, cout), out_dtype),
                   jax.ShapeDtypeStruct((HT, 1, cout), jnp.float32),
                   jax.ShapeDtypeStruct((HT, 1, cout), jnp.float32)),
        grid=(HT,),
        in_specs=[pl.BlockSpec((N, 1, th + 2, W + 2, C), lambda h: (0, h, 0, 0, 0)),
                  pl.BlockSpec((3, 3 * cin, cout), lambda h: (0, 0, 0)),
                  pl.BlockSpec((1, cout), lambda h: (0, 0))],
        out_specs=(pl.BlockSpec((N, 1, th, W, cout), lambda h: (0, h, 0, 0, 0)),
                   pl.BlockSpec((1, 1, cout), lambda h: (h, 0, 0)),
                   pl.BlockSpec((1, 1, cout), lambda h: (h, 0, 0))),
        scratch_shapes=[pltpu.VMEM((N, th + 2, W, 3 * C), jnp.bfloat16),
                        pltpu.VMEM((M, cout), jnp.float32)],
        compiler_params=pltpu.CompilerParams(
            dimension_semantics=("parallel",),
            vmem_limit_bytes=_VMEM_LIMIT),
    )(a, wm, bm)

    y = out.reshape(N, H, W, cout)
    s = jnp.sum(psum[:, 0, :], axis=0)
    ss = jnp.sum(psq[:, 0, :], axis=0)
    return y, (s, ss, N * H * W)


def _im2col(x, kh, kw, stride, pt, pb, pl_, pr):
    x = jnp.pad(x, ((0, 0), (pt, pb), (pl_, pr), (0, 0)))
    N, H, W, C = x.shape
    Ho = (H - kh) // stride + 1
    Wo = (W - kw) // stride + 1
    cols = []
    for i in range(kh):
        for j in range(kw):
            cols.append(x[:, i:i + (Ho - 1) * stride + 1:stride,
                            j:j + (Wo - 1) * stride + 1:stride, :])
    patches = jnp.stack(cols, axis=-2)
    return patches.reshape(N * Ho * Wo, kh * kw * C), (N, Ho, Wo)


def conv2d(x, p, stride=1, padding=(0, 0), out_dtype=jnp.bfloat16):
    w, b = p['w'], p['b']
    cout, cin, kh, kw = w.shape
    N, H, W, _ = x.shape
    if kh == 1 and kw == 1 and stride == 1:
        mat = x.reshape(N * H * W, cin)
        wmat = jnp.transpose(w, (2, 3, 1, 0)).reshape(cin, cout)
        out, s, ss = matmul_bias_stats(mat, wmat, b, out_dtype=out_dtype)
        return out.reshape(N, H, W, cout), (s, ss, N * H * W)
    if kh == 3 and kw == 3 and stride == 1 and padding == (1, 1) and H % 8 == 0:
        if H >= 64:
            return conv3x3_direct_c(x, p, out_dtype=out_dtype)
        return conv3x3_direct_b(x, p, out_dtype=out_dtype)
    if stride == 2:
        xpad = jnp.pad(x.astype(jnp.bfloat16),
                       ((0, 0), (padding[0], padding[0]),
                        (padding[1], padding[1]), (0, 0)))
        Np, Hp, Wp, C = xpad.shape
        Ho = (Hp - kh) // 2 + 1
        Wo = (Wp - kw) // 2 + 1
        rows = jnp.concatenate(
            [xpad[:, i:i + 2 * (Ho - 1) + 1:2, :, :] for i in range(kh)],
            axis=-1) if kh > 1 else xpad[:, ::2]
        cols = jnp.concatenate(
            [rows[:, :, j:j + 2 * (Wo - 1) + 1:2, :] for j in range(kw)],
            axis=-1) if kw > 1 else rows[:, :, ::2]
        patches = cols.reshape(Np * Ho * Wo, kw * kh * cin)
        wmat = jnp.transpose(w, (3, 2, 1, 0)).reshape(kh * kw * cin, cout)
        out, s, ss = matmul_bias_stats(patches, wmat, b, out_dtype=out_dtype)
        return out.reshape(Np, Ho, Wo, cout), (s, ss, Np * Ho * Wo)
    patches, (N, Ho, Wo) = _im2col(x.astype(jnp.bfloat16), kh, kw, stride,
                                   padding[0], padding[0], padding[1], padding[1])
    wmat = jnp.transpose(w, (2, 3, 1, 0)).reshape(kh * kw * cin, cout)
    out, s, ss = matmul_bias_stats(patches, wmat, b, out_dtype=out_dtype)
    return out.reshape(N, Ho, Wo, cout), (s, ss, N * Ho * Wo)


def _s2_taps(k, pad):
    taps = {0: [], 1: []}
    for d in (0, 1):
        for i in range(k):
            if (d + pad - i) % 2 == 0:
                taps[d].append(((d + pad - i) // 2, i))
    ts = sorted({t for d in (0, 1) for (t, _) in taps[d]})
    return taps, ts


def conv_transpose2d(x, p, padding=(0, 0), output_padding=(0, 0),
                     out_dtype=jnp.bfloat16):
    w, b = p['w'], p['b']
    cin, cout, kh, kw = w.shape
    N, H, W, _ = x.shape
    Ho = 2 * H
    Wo = 2 * W

    taps_h, ts_h = _s2_taps(kh, padding[0])
    taps_w, ts_w = _s2_taps(kw, padding[1])
    ph_lo, ph_hi = max(0, -ts_h[0]), max(0, ts_h[-1])
    pw_lo, pw_hi = max(0, -ts_w[0]), max(0, ts_w[-1])

    if len(ts_h) == 1 and len(ts_w) == 1 and ts_h[0] == 0 and ts_w[0] == 0:
        patches = x.astype(jnp.bfloat16).reshape(N * H * W, cin)
    else:
        xp = jnp.pad(x.astype(jnp.bfloat16),
                     ((0, 0), (ph_lo, ph_hi), (pw_lo, pw_hi), (0, 0)))
        cols = []
        for th in ts_h:
            for tw in ts_w:
                cols.append(xp[:, th + ph_lo: th + ph_lo + H,
                                 tw + pw_lo: tw + pw_lo + W, :])
        patches = jnp.concatenate(cols, axis=-1).reshape(
            N * H * W, len(ts_h) * len(ts_w) * cin)

    ih_tbl = np.full((len(ts_h), 2), -1, np.int64)
    for d in (0, 1):
        for (t, i) in taps_h[d]:
            ih_tbl[ts_h.index(t), d] = i
    iw_tbl = np.full((len(ts_w), 2), -1, np.int64)
    for d in (0, 1):
        for (t, i) in taps_w[d]:
            iw_tbl[ts_w.index(t), d] = i

    zeros = jnp.zeros((cin, cout), w.dtype)
    rows = []
    for a_ in range(len(ts_h)):
        for b_ in range(len(ts_w)):
            phase_cols = []
            for dh in (0, 1):
                for dw in (0, 1):
                    ih, iw = int(ih_tbl[a_, dh]), int(iw_tbl[b_, dw])
                    phase_cols.append(zeros if (ih < 0 or iw < 0) else w[:, :, ih, iw])
            rows.append(jnp.concatenate(phase_cols, axis=1))
    wmat = jnp.concatenate(rows, axis=0)

    out, s, ss = matmul_bias_stats(patches, wmat, jnp.tile(b, 4),
                                   out_dtype=out_dtype)
    out = out.reshape(N, H, W, 2, 2, cout)
    out = out.transpose(0, 1, 3, 2, 4, 5).reshape(N, 2 * H, 2 * W, cout)
    stats = (s.reshape(4, cout).sum(0), ss.reshape(4, cout).sum(0), N * Ho * Wo)
    return out, stats


def maxpool_3x3_s2_p1(x):
    xp = jnp.pad(x, ((0, 0), (1, 1), (1, 1), (0, 0)), constant_values=-jnp.inf)
    return lax.reduce_window(xp, -jnp.inf, lax.max, (1, 3, 3, 1), (1, 2, 2, 1),
                             'VALID')


def upsample_nearest(x, scale):
    if scale == 1:
        return x
    return jnp.repeat(jnp.repeat(x, scale, axis=1), scale, axis=2)


def conv_bn(x, cp, bnp, stride=1, padding=(0, 0), relu=True,
            residual=None, post_add=None):
    y, stats = conv2d(x, cp, stride=stride, padding=padding)
    return batchnorm2d(y, bnp, stats, relu=relu, residual=residual,
                       post_add=post_add)


def convT_bn(x, cp, bnp, padding=(0, 0), output_padding=(0, 0)):
    y, stats = conv_transpose2d(x, cp, padding=padding,
                                output_padding=output_padding)
    return batchnorm2d(y, bnp, stats, relu=True)


def basic_block(x, p, post_add=None):
    x = conv_bn(x, p['conv1'], p['bn1'], padding=(1, 1), relu=True)
    x = conv_bn(x, p['conv2'], p['bn2'], padding=(1, 1), relu=True,
                post_add=post_add)
    return x


def resnet_block(x, p, stride=1):
    identity = x
    out = conv_bn(x, p['conv1'], p['bn1'], stride=stride, padding=(1, 1),
                  relu=True)
    y, stats = conv2d(out, p['conv2'], stride=1, padding=(1, 1))
    if 'down_conv' in p:
        identity = conv_bn(x, p['down_conv'], p['down_bn'],
                           stride=stride, padding=(0, 0), relu=False)
    return batchnorm2d(y, p['bn2'], stats, relu=True, residual=identity)


def encoder(x, p):
    x = conv_bn(x, p['conv1'], p['bn1'], stride=2, padding=(3, 3), relu=True)
    x = maxpool_3x3_s2_p1(x)
    x = resnet_block(x, p['layer1'][0])
    x = resnet_block(x, p['layer1'][1])
    x1 = x
    x = resnet_block(x, p['layer2'][0], stride=2)
    x = resnet_block(x, p['layer2'][1])
    x2 = x
    x = resnet_block(x, p['layer3'][0], stride=2)
    x = resnet_block(x, p['layer3'][1])
    x3 = x
    x = resnet_block(x, p['layer4'][0], stride=2)
    x = resnet_block(x, p['layer4'][1])
    x4 = x
    return x1, x2, x3, x4


def upsample_module(x, p, padding=(0, 0), output_padding=(0, 0), skip=None):
    x = convT_bn(x, p['convT'], p['bn'], padding, output_padding)
    x = basic_block(x, p['conv'], post_add=skip)
    return x


def fpn_module(x, p, scale):
    x = conv_bn(x, p['conv1'], p['bn'], padding=(1, 1), relu=True)
    return upsample_nearest(x, scale)


def row_detection(x, p):
    x1, x2, x3, x4 = encoder(x, p['down'])
    x = upsample_module(x4, p['up1'], padding=(1, 1), output_padding=(1, 1),
                        skip=x3)
    x3 = x
    x = upsample_module(x, p['up2'], padding=(1, 1), output_padding=(1, 1),
                        skip=x2)
    x2 = x
    x = upsample_module(x, p['up3'], padding=(1, 1), output_padding=(1, 1),
                        skip=x1)
    x1 = x
    x = upsample_module(x, p['up4'], padding=(1, 1), output_padding=(1, 1))
    x3 = fpn_module(x3, p['fpn1'], 8)
    x2 = fpn_module(x2, p['fpn2'], 4)
    x1 = fpn_module(x1, p['fpn3'], 2)
    x = fpn_module(x, p['fpn4'], 1)
    x = jnp.concatenate([x, x1, x2, x3], axis=-1)
    x = upsample_module(x, p['up5'])
    x = basic_block(x, p['conv1'])
    y, _ = conv2d(x, p['conv2'], padding=(0, 0), out_dtype=jnp.float32)
    return y


class _ParamGen:
    def __init__(self, seed=0):
        self.key = jax.random.PRNGKey(seed)

    def next(self):
        self.key, sub = jax.random.split(self.key)
        return sub


def _make_conv(pg, cin, cout, k, bias=True):
    kh, kw = (k, k) if isinstance(k, int) else k
    w = jax.random.normal(pg.next(), (cout, cin, kh, kw), jnp.float32) / np.sqrt(cin * kh * kw)
    b = (jax.random.normal(pg.next(), (cout,), jnp.float32) * 0.01 if bias
         else jnp.zeros((cout,), jnp.float32))
    return {'w': w, 'b': b}


def _make_convT(pg, cin, cout, k):
    kh, kw = k
    w = jax.random.normal(pg.next(), (cin, cout, kh, kw), jnp.float32) / np.sqrt(cin * kh * kw)
    b = jax.random.normal(pg.next(), (cout,), jnp.float32) * 0.01
    return {'w': w, 'b': b}


def _make_bn(c):
    return {'gamma': jnp.ones((c,), jnp.float32), 'beta': jnp.zeros((c,), jnp.float32)}


def _make_resnet_block(pg, cin, cout, stride=1):
    p = {'conv1': _make_conv(pg, cin, cout, 3, bias=False), 'bn1': _make_bn(cout),
         'conv2': _make_conv(pg, cout, cout, 3, bias=False), 'bn2': _make_bn(cout)}
    if stride != 1 or cin != cout:
        p['down_conv'] = _make_conv(pg, cin, cout, 1, bias=False)
        p['down_bn'] = _make_bn(cout)
    return p


def _make_basic_block(pg, cin, cout):
    return {'conv1': _make_conv(pg, cin, cout, 3), 'bn1': _make_bn(cout),
            'conv2': _make_conv(pg, cout, cout, 3), 'bn2': _make_bn(cout)}


def _make_upsample(pg, cin, cout, k=(3, 3)):
    return {'convT': _make_convT(pg, cin, cout, k), 'bn': _make_bn(cout),
            'conv': _make_basic_block(pg, cout, cout)}


def _make_fpn(pg, cin):
    return {'conv1': _make_conv(pg, cin, 64, 3), 'bn': _make_bn(64)}


def _make_params(out_channel=2):
    pg = _ParamGen(0)
    down = {'conv1': _make_conv(pg, 3, 64, 7, bias=False), 'bn1': _make_bn(64),
            'layer1': [_make_resnet_block(pg, 64, 64), _make_resnet_block(pg, 64, 64)],
            'layer2': [_make_resnet_block(pg, 64, 128, 2), _make_resnet_block(pg, 128, 128)],
            'layer3': [_make_resnet_block(pg, 128, 256, 2), _make_resnet_block(pg, 256, 256)],
            'layer4': [_make_resnet_block(pg, 256, 512, 2), _make_resnet_block(pg, 512, 512)]}
    return {'down': down,
            'up1': _make_upsample(pg, 512, 256), 'up2': _make_upsample(pg, 256, 128),
            'up3': _make_upsample(pg, 128, 64), 'up4': _make_upsample(pg, 64, 64),
            'fpn1': _make_fpn(pg, 256), 'fpn2': _make_fpn(pg, 128),
            'fpn3': _make_fpn(pg, 64), 'fpn4': _make_fpn(pg, 64),
            'up5': _make_upsample(pg, 256, 64, k=(2, 2)),
            'conv1': _make_basic_block(pg, 64, 32),
            'conv2': _make_conv(pg, 32, out_channel, 1)}


_TREEDEF = None


def _treedef():
    global _TREEDEF
    if _TREEDEF is None:
        _, _TREEDEF = jax.tree_util.tree_flatten(_make_params(2))
    return _TREEDEF


def kernel(x_nchw, *leaves):
    params = jax.tree_util.tree_unflatten(_treedef(), list(leaves))
    x = jnp.transpose(x_nchw, (0, 2, 3, 1))
    y = row_detection(x, params)
    return jnp.transpose(y, (0, 3, 1, 2))
